# baseline (speedup 1.0000x reference)
import functools

import jax
import jax.numpy as jnp
from jax import lax
from jax.experimental import pallas as pl
from jax.experimental.pallas import tpu as pltpu
from jax.experimental.pallas import tpu_sc as plsc

N_NODES = 10000
N_EDGES = 160000
F_IN = 32
F_H = 32
F_EF = 16
N_G = 64

NP = 10240
EP = 163840

NC = 2
NS = 16
NW = NC * NS
EPW = EP // NW
CHUNK = 128
NCH = EPW // CHUNK
RPT = NP // NS

_SC_MESH = dict(core_axis_name="c", subcore_axis_name="s")


def _sc_gather(table, idx_r):
    F = table.shape[1]

    @functools.partial(
        pl.kernel,
        out_type=jax.ShapeDtypeStruct((EP, F), jnp.float32),
        scratch_types=[
            pltpu.VMEM((NCH, CHUNK), jnp.int32),
            pltpu.VMEM((CHUNK, F), jnp.float32),
            pltpu.SemaphoreType.DMA,
        ],
        mesh=plsc.VectorSubcoreMesh(**_SC_MESH),
        compiler_params=pltpu.CompilerParams(use_tc_tiling_on_sc=False),
    )
    def gk(table_hbm, idx_hbm, out_hbm, idx_v, rows_v, sem):
        cid = lax.axis_index("c")
        sid = lax.axis_index("s")
        wid = cid * NS + sid
        base = wid * EPW
        pltpu.sync_copy(idx_hbm.at[wid], idx_v)

        def body(j, carry):
            off = pl.multiple_of(base + j * CHUNK, 8)
            pltpu.async_copy(table_hbm.at[idx_v.at[j]], rows_v, sem).wait()
            pltpu.sync_copy(rows_v, out_hbm.at[pl.ds(off, CHUNK)])
            return carry

        lax.fori_loop(0, NCH, body, 0)

    return gk(table, idx_r)


def _sc_scatter(msg, dst_r, zeros2d, zeros1d, ones1d, with_cnt):
    F = msg.shape[1]
    outs = [jax.ShapeDtypeStruct((NC * NP, F), jnp.float32)]
    scratch = [
        pltpu.VMEM((NCH, CHUNK), jnp.int32),
        pltpu.VMEM((CHUNK, F), jnp.float32),
        pltpu.VMEM_SHARED((NP, F), jnp.float32),
    ]
    if with_cnt:
        outs.append(jax.ShapeDtypeStruct((NC * NP,), jnp.float32))
        scratch += [
            pltpu.VMEM((CHUNK,), jnp.float32),
            pltpu.VMEM_SHARED((NP,), jnp.float32),
        ]

    @functools.partial(
        pl.kernel,
        out_type=tuple(outs) if with_cnt else outs[0],
        scratch_types=scratch,
        mesh=plsc.VectorSubcoreMesh(**_SC_MESH),
        compiler_params=pltpu.CompilerParams(use_tc_tiling_on_sc=False),
    )
    def sk(msg_hbm, dst_hbm, z2_hbm, z1_hbm, ones_hbm, *refs):
        if with_cnt:
            acc_hbm, cnt_hbm, idx_v, msg_v, acc_sh, ones_v, cnt_sh = refs
        else:
            acc_hbm, idx_v, msg_v, acc_sh = refs
        cid = lax.axis_index("c")
        sid = lax.axis_index("s")
        wid = cid * NS + sid
        base = wid * EPW
        zoff = pl.multiple_of(sid * RPT, 8)

        pltpu.sync_copy(z2_hbm.at[pl.ds(zoff, RPT)],
                        acc_sh.at[pl.ds(zoff, RPT)])
        if with_cnt:
            @pl.when(sid == 0)
            def _():
                pltpu.sync_copy(z1_hbm, cnt_sh)
            pltpu.sync_copy(ones_hbm, ones_v)
        pltpu.sync_copy(dst_hbm.at[wid], idx_v)
        plsc.subcore_barrier()

        def body(j, carry):
            off = pl.multiple_of(base + j * CHUNK, 8)
            pltpu.sync_copy(msg_hbm.at[pl.ds(off, CHUNK)], msg_v)
            pltpu.sync_copy(msg_v, acc_sh.at[idx_v.at[j]], add=True)
            if with_cnt:
                pltpu.sync_copy(ones_v, cnt_sh.at[idx_v.at[j]], add=True)
            return carry

        lax.fori_loop(0, NCH, body, 0)
        plsc.subcore_barrier()

        doff = pl.multiple_of(cid * NP + sid * RPT, 8)
        pltpu.sync_copy(acc_sh.at[pl.ds(zoff, RPT)],
                        acc_hbm.at[pl.ds(doff, RPT)])
        if with_cnt:
            @pl.when(sid == 0)
            def _():
                coff = pl.multiple_of(cid * NP, 8)
                pltpu.sync_copy(cnt_sh, cnt_hbm.at[pl.ds(coff, NP)])

    return sk(msg, dst_r, zeros2d, zeros1d, ones1d)


BE = 1024


def _msg_body(ea_ref, xs_ref, Wa_ref, ba_ref, W2w_ref, R_ref, S_ref, Bb_ref,
              out_ref):
    h = jnp.maximum(
        jnp.dot(ea_ref[...], Wa_ref[...],
                preferred_element_type=jnp.float32, precision=lax.Precision.HIGHEST) + ba_ref[...], 0.0)
    T = jnp.dot(xs_ref[...], W2w_ref[...], preferred_element_type=jnp.float32, precision=lax.Precision.HIGHEST)
    Hx = jnp.dot(h, R_ref[...], preferred_element_type=jnp.float32, precision=lax.Precision.HIGHEST)
    msg = jnp.dot(T * Hx, S_ref[...], preferred_element_type=jnp.float32, precision=lax.Precision.HIGHEST)
    msg = msg + jnp.dot(xs_ref[...], Bb_ref[...],
                        preferred_element_type=jnp.float32, precision=lax.Precision.HIGHEST)
    out_ref[...] = msg


def _msg_call(ea, xs, Wa, ba, W2w, R, S, Bb, interpret=False):
    din = xs.shape[1]
    dk = W2w.shape[1]
    return pl.pallas_call(
        _msg_body,
        grid=(EP // BE,),
        in_specs=[
            pl.BlockSpec((BE, F_EF), lambda i: (i, 0)),
            pl.BlockSpec((BE, din), lambda i: (i, 0)),
            pl.BlockSpec((F_EF, F_H), lambda i: (0, 0)),
            pl.BlockSpec((1, F_H), lambda i: (0, 0)),
            pl.BlockSpec((din, dk), lambda i: (0, 0)),
            pl.BlockSpec((F_H, dk), lambda i: (0, 0)),
            pl.BlockSpec((dk, F_H), lambda i: (0, 0)),
            pl.BlockSpec((din, F_H), lambda i: (0, 0)),
        ],
        out_specs=pl.BlockSpec((BE, F_H), lambda i: (i, 0)),
        out_shape=jax.ShapeDtypeStruct((EP, F_H), jnp.float32),
        interpret=interpret,
    )(ea, xs, Wa, ba, W2w, R, S, Bb)


BN = 2048
NBN = NP // BN


def _node1_body(acc0_ref, acc1_ref, cnt0_ref, cnt1_ref, x_ref, root_ref,
                bias_ref, out_ref):
    c = jnp.maximum(cnt0_ref[...] + cnt1_ref[...], 1.0)
    a = (acc0_ref[...] + acc1_ref[...]) / c
    t = a + jnp.dot(x_ref[...], root_ref[...],
                    preferred_element_type=jnp.float32, precision=lax.Precision.HIGHEST) + bias_ref[...]
    out_ref[...] = jnp.where(t > 0.0, t, jnp.exp(t) - 1.0)


def _node1_call(acc, cnt2d, x, root, bias, interpret=False):
    din = x.shape[1]
    return pl.pallas_call(
        _node1_body,
        grid=(NBN,),
        in_specs=[
            pl.BlockSpec((BN, F_H), lambda i: (i, 0)),
            pl.BlockSpec((BN, F_H), lambda i: (i + NBN, 0)),
            pl.BlockSpec((BN, 1), lambda i: (i, 0)),
            pl.BlockSpec((BN, 1), lambda i: (i + NBN, 0)),
            pl.BlockSpec((BN, din), lambda i: (i, 0)),
            pl.BlockSpec((din, F_H), lambda i: (0, 0)),
            pl.BlockSpec((1, F_H), lambda i: (0, 0)),
        ],
        out_specs=pl.BlockSpec((BN, F_H), lambda i: (i, 0)),
        out_shape=jax.ShapeDtypeStruct((NP, F_H), jnp.float32),
        interpret=interpret,
    )(acc, acc, cnt2d, cnt2d, x, root, bias)


def _node2_body(acc0_ref, acc1_ref, cnt0_ref, cnt1_ref, h1_ref, b_ref,
                root_ref, bias_ref, Wp1_ref, bp1_ref, Wp2_ref, bp2_ref,
                Wp3_ref, bp3_ref, out_o_ref, out_r_ref, accg, cntg):
    i = pl.program_id(0)

    @pl.when(i == 0)
    def _():
        accg[...] = jnp.zeros_like(accg)
        cntg[...] = jnp.zeros_like(cntg)

    c = jnp.maximum(cnt0_ref[...] + cnt1_ref[...], 1.0)
    a = (acc0_ref[...] + acc1_ref[...]) / c
    t = a + jnp.dot(h1_ref[...], root_ref[...],
                    preferred_element_type=jnp.float32, precision=lax.Precision.HIGHEST) + bias_ref[...]
    h2 = jnp.where(t > 0.0, t, jnp.exp(t) - 1.0)

    ids = b_ref[0]
    onehot = (lax.broadcasted_iota(jnp.int32, (N_G, BN), 0)
              == ids).astype(jnp.float32)
    accg[...] += jnp.dot(onehot, h2, preferred_element_type=jnp.float32, precision=lax.Precision.HIGHEST)
    cntg[...] += jnp.broadcast_to(
        jnp.sum(onehot, axis=1, keepdims=True), (N_G, F_H))

    @pl.when(i == NBN - 1)
    def _():
        r = accg[...] / jnp.maximum(cntg[...], 1.0)
        o = jnp.maximum(
            jnp.dot(r, Wp1_ref[...],
                    preferred_element_type=jnp.float32, precision=lax.Precision.HIGHEST) + bp1_ref[...], 0.0)
        o = jnp.maximum(
            jnp.dot(o, Wp2_ref[...],
                    preferred_element_type=jnp.float32, precision=lax.Precision.HIGHEST) + bp2_ref[...], 0.0)
        o3 = jnp.dot(o, Wp3_ref[...],
                     preferred_element_type=jnp.float32, precision=lax.Precision.HIGHEST) + bp3_ref[...]
        out_o_ref[...] = o3
        out_r_ref[...] = r


def _node2_call(acc, cnt2d, h1, batch3, rootl, biasl, Wp1, bp1, Wp2, bp2,
                Wp3, bp3, interpret=False):
    return pl.pallas_call(
        _node2_body,
        grid=(NBN,),
        in_specs=[
            pl.BlockSpec((BN, F_H), lambda i: (i, 0)),
            pl.BlockSpec((BN, F_H), lambda i: (i + NBN, 0)),
            pl.BlockSpec((BN, 1), lambda i: (i, 0)),
            pl.BlockSpec((BN, 1), lambda i: (i + NBN, 0)),
            pl.BlockSpec((BN, F_H), lambda i: (i, 0)),
            pl.BlockSpec((1, 1, BN), lambda i: (i, 0, 0)),
            pl.BlockSpec((F_H, F_H), lambda i: (0, 0)),
            pl.BlockSpec((1, F_H), lambda i: (0, 0)),
            pl.BlockSpec((F_H, F_H), lambda i: (0, 0)),
            pl.BlockSpec((1, F_H), lambda i: (0, 0)),
            pl.BlockSpec((F_H, F_H // 2), lambda i: (0, 0)),
            pl.BlockSpec((1, F_H // 2), lambda i: (0, 0)),
            pl.BlockSpec((F_H // 2, 1), lambda i: (0, 0)),
            pl.BlockSpec((1, 1), lambda i: (0, 0)),
        ],
        out_specs=[
            pl.BlockSpec((N_G, 1), lambda i: (0, 0)),
            pl.BlockSpec((N_G, F_H), lambda i: (0, 0)),
        ],
        out_shape=[
            jax.ShapeDtypeStruct((N_G, 1), jnp.float32),
            jax.ShapeDtypeStruct((N_G, F_H), jnp.float32),
        ],
        scratch_shapes=[
            pltpu.VMEM((N_G, F_H), jnp.float32),
            pltpu.VMEM((N_G, F_H), jnp.float32),
        ],
        interpret=interpret,
    )(acc, acc, cnt2d, cnt2d, h1, batch3, rootl, biasl, Wp1, bp1, Wp2, bp2,
      Wp3, bp3)


def _prep_w2w(Wb, din, dout):
    return jnp.transpose(Wb.reshape(F_H, din, dout), (1, 0, 2)).reshape(
        din, F_H * dout)


def kernel(x, edge_index, edge_attr, batch, W1a, b1a, W1b, b1b, root1, bias1,
           Wla, bla, Wlb, blb, rootl, biasl, Wp1, bp1, Wp2, bp2, Wp3, bp3):
    x = jnp.pad(x.astype(jnp.float32), ((0, NP - N_NODES), (0, 0)))
    ea = jnp.pad(edge_attr.astype(jnp.float32),
                 ((0, EP - N_EDGES), (0, 0)))
    src = edge_index[0].astype(jnp.int32)
    dst = edge_index[1].astype(jnp.int32)
    pad_e = EP - N_EDGES
    src_r = jnp.concatenate(
        [src, jnp.zeros((pad_e,), jnp.int32)]).reshape(NW, NCH, CHUNK)
    dst_r = jnp.concatenate(
        [dst, jnp.full((pad_e,), N_NODES, jnp.int32)]).reshape(NW, NCH, CHUNK)
    batch3 = jnp.concatenate(
        [batch.astype(jnp.int32),
         jnp.full((NP - N_NODES,), N_G, jnp.int32)]).reshape(NBN, 1, BN)

    eye = jnp.eye(F_H, dtype=jnp.float32)
    R = jnp.repeat(eye, F_H, axis=1)
    S = jnp.tile(eye, (F_H, 1))
    z2 = jnp.zeros((NP, F_H), jnp.float32)
    z1 = jnp.zeros((NP,), jnp.float32)
    ones = jnp.ones((CHUNK,), jnp.float32)

    xs = _sc_gather(x, src_r)
    msg1 = _msg_call(ea, xs, W1a, b1a.reshape(1, F_H),
                     _prep_w2w(W1b, F_IN, F_H), R, S,
                     b1b.reshape(F_IN, F_H))
    acc1, cnt = _sc_scatter(msg1, dst_r, z2, z1, ones, with_cnt=True)
    cnt2d = cnt.reshape(NC * NP, 1)
    h1 = _node1_call(acc1, cnt2d, x, root1, bias1.reshape(1, F_H))

    hs = _sc_gather(h1, src_r)
    msg2 = _msg_call(ea, hs, Wla, bla.reshape(1, F_H),
                     _prep_w2w(Wlb, F_H, F_H), R, S,
                     blb.reshape(F_H, F_H))
    acc2 = _sc_scatter(msg2, dst_r, z2, z1, ones, with_cnt=False)
    out_o, out_r = _node2_call(
        acc2, cnt2d, h1, batch3, rootl, biasl.reshape(1, F_H), Wp1,
        bp1.reshape(1, F_H), Wp2, bp2.reshape(1, F_H // 2), Wp3,
        bp3.reshape(1, 1))
    return out_o.reshape(N_G), out_r


# speedup vs baseline: 1.4008x; 1.4008x over previous
import functools

import jax
import jax.numpy as jnp
from jax import lax
from jax.experimental import pallas as pl
from jax.experimental.pallas import tpu as pltpu
from jax.experimental.pallas import tpu_sc as plsc

N_NODES = 10000
N_EDGES = 160000
F_IN = 32
F_H = 32
F_EF = 16
N_G = 64

NP = 10240
EP = 163840

NC = 2
NS = 16
NW = NC * NS
EPW = EP // NW
CHUNK = 128
NCH = EPW // CHUNK
RPT = NP // NS

_SC_MESH = dict(core_axis_name="c", subcore_axis_name="s")


def _sc_gather(table, idx_r):
    F = table.shape[1]

    @functools.partial(
        pl.kernel,
        out_type=jax.ShapeDtypeStruct((EP, F), jnp.float32),
        scratch_types=[
            pltpu.VMEM((NCH, CHUNK), jnp.int32),
            pltpu.VMEM((CHUNK, F), jnp.float32),
            pltpu.SemaphoreType.DMA,
        ],
        mesh=plsc.VectorSubcoreMesh(**_SC_MESH),
        compiler_params=pltpu.CompilerParams(use_tc_tiling_on_sc=False),
    )
    def gk(table_hbm, idx_hbm, out_hbm, idx_v, rows_v, sem):
        cid = lax.axis_index("c")
        sid = lax.axis_index("s")
        wid = cid * NS + sid
        base = wid * EPW
        pltpu.sync_copy(idx_hbm.at[wid], idx_v)

        def body(j, carry):
            off = pl.multiple_of(base + j * CHUNK, 8)
            pltpu.async_copy(table_hbm.at[idx_v.at[j]], rows_v, sem).wait()
            pltpu.sync_copy(rows_v, out_hbm.at[pl.ds(off, CHUNK)])
            return carry

        lax.fori_loop(0, NCH, body, 0)

    return gk(table, idx_r)


def _sc_scatter(msg, dst_r, zeros2d, zeros1d, ones1d, with_cnt):
    F = msg.shape[1]
    outs = [jax.ShapeDtypeStruct((NC * NP, F), jnp.float32)]
    scratch = [
        pltpu.VMEM((NCH, CHUNK), jnp.int32),
        pltpu.VMEM((CHUNK, F), jnp.float32),
        pltpu.VMEM_SHARED((NP, F), jnp.float32),
    ]
    if with_cnt:
        outs.append(jax.ShapeDtypeStruct((NC * NP,), jnp.float32))
        scratch += [
            pltpu.VMEM((CHUNK,), jnp.float32),
            pltpu.VMEM_SHARED((NP,), jnp.float32),
        ]

    @functools.partial(
        pl.kernel,
        out_type=tuple(outs) if with_cnt else outs[0],
        scratch_types=scratch,
        mesh=plsc.VectorSubcoreMesh(**_SC_MESH),
        compiler_params=pltpu.CompilerParams(use_tc_tiling_on_sc=False),
    )
    def sk(msg_hbm, dst_hbm, z2_hbm, z1_hbm, ones_hbm, *refs):
        if with_cnt:
            acc_hbm, cnt_hbm, idx_v, msg_v, acc_sh, ones_v, cnt_sh = refs
        else:
            acc_hbm, idx_v, msg_v, acc_sh = refs
        cid = lax.axis_index("c")
        sid = lax.axis_index("s")
        wid = cid * NS + sid
        base = wid * EPW
        zoff = pl.multiple_of(sid * RPT, 8)

        pltpu.sync_copy(z2_hbm.at[pl.ds(zoff, RPT)],
                        acc_sh.at[pl.ds(zoff, RPT)])
        if with_cnt:
            @pl.when(sid == 0)
            def _():
                pltpu.sync_copy(z1_hbm, cnt_sh)
            pltpu.sync_copy(ones_hbm, ones_v)
        pltpu.sync_copy(dst_hbm.at[wid], idx_v)
        plsc.subcore_barrier()

        def body(j, carry):
            off = pl.multiple_of(base + j * CHUNK, 8)
            pltpu.sync_copy(msg_hbm.at[pl.ds(off, CHUNK)], msg_v)
            pltpu.sync_copy(msg_v, acc_sh.at[idx_v.at[j]], add=True)
            if with_cnt:
                pltpu.sync_copy(ones_v, cnt_sh.at[idx_v.at[j]], add=True)
            return carry

        lax.fori_loop(0, NCH, body, 0)
        plsc.subcore_barrier()

        doff = pl.multiple_of(cid * NP + sid * RPT, 8)
        pltpu.sync_copy(acc_sh.at[pl.ds(zoff, RPT)],
                        acc_hbm.at[pl.ds(doff, RPT)])
        if with_cnt:
            @pl.when(sid == 0)
            def _():
                coff = pl.multiple_of(cid * NP, 8)
                pltpu.sync_copy(cnt_sh, cnt_hbm.at[pl.ds(coff, NP)])

    return sk(msg, dst_r, zeros2d, zeros1d, ones1d)


BE = 1024


def _msg_body(ea_ref, xs_ref, Wa_ref, ba_ref, W2w_ref, S_ref, Bb_ref,
              out_ref):
    h = jnp.maximum(
        jnp.dot(ea_ref[...], Wa_ref[...],
                preferred_element_type=jnp.float32,
                precision=lax.Precision.HIGHEST) + ba_ref[...], 0.0)
    T = jnp.dot(xs_ref[...], W2w_ref[...],
                preferred_element_type=jnp.float32,
                precision=lax.Precision.HIGHEST)
    Hxt = jnp.concatenate([h] * F_H, axis=1)
    msg = jnp.dot(T * Hxt, S_ref[...], preferred_element_type=jnp.float32,
                  precision=lax.Precision.HIGHEST)
    msg = msg + jnp.dot(xs_ref[...], Bb_ref[...],
                        preferred_element_type=jnp.float32,
                        precision=lax.Precision.HIGHEST)
    out_ref[...] = msg


def _msg_call(ea, xs, Wa, ba, Wb, bb, interpret=False):
    din = xs.shape[1]
    dk = F_H * F_H
    W2w = jnp.transpose(Wb.reshape(F_H, din, F_H), (1, 2, 0)).reshape(din, dk)
    S = jnp.kron(jnp.eye(F_H, dtype=jnp.float32),
                 jnp.ones((F_H, 1), jnp.float32))
    Bb = bb.reshape(din, F_H)
    return pl.pallas_call(
        _msg_body,
        grid=(EP // BE,),
        in_specs=[
            pl.BlockSpec((BE, F_EF), lambda i: (i, 0)),
            pl.BlockSpec((BE, din), lambda i: (i, 0)),
            pl.BlockSpec((F_EF, F_H), lambda i: (0, 0)),
            pl.BlockSpec((1, F_H), lambda i: (0, 0)),
            pl.BlockSpec((din, dk), lambda i: (0, 0)),
            pl.BlockSpec((dk, F_H), lambda i: (0, 0)),
            pl.BlockSpec((din, F_H), lambda i: (0, 0)),
        ],
        out_specs=pl.BlockSpec((BE, F_H), lambda i: (i, 0)),
        out_shape=jax.ShapeDtypeStruct((EP, F_H), jnp.float32),
        interpret=interpret,
    )(ea, xs, Wa, ba.reshape(1, F_H), W2w, S, Bb)


BN = 2048
NBN = NP // BN


def _node1_body(acc0_ref, acc1_ref, cnt0_ref, cnt1_ref, x_ref, root_ref,
                bias_ref, out_ref):
    c = jnp.maximum(cnt0_ref[...] + cnt1_ref[...], 1.0)
    a = (acc0_ref[...] + acc1_ref[...]) / c
    t = a + jnp.dot(x_ref[...], root_ref[...],
                    preferred_element_type=jnp.float32, precision=lax.Precision.HIGHEST) + bias_ref[...]
    out_ref[...] = jnp.where(t > 0.0, t, jnp.exp(t) - 1.0)


def _node1_call(acc, cnt2d, x, root, bias, interpret=False):
    din = x.shape[1]
    return pl.pallas_call(
        _node1_body,
        grid=(NBN,),
        in_specs=[
            pl.BlockSpec((BN, F_H), lambda i: (i, 0)),
            pl.BlockSpec((BN, F_H), lambda i: (i + NBN, 0)),
            pl.BlockSpec((BN, 1), lambda i: (i, 0)),
            pl.BlockSpec((BN, 1), lambda i: (i + NBN, 0)),
            pl.BlockSpec((BN, din), lambda i: (i, 0)),
            pl.BlockSpec((din, F_H), lambda i: (0, 0)),
            pl.BlockSpec((1, F_H), lambda i: (0, 0)),
        ],
        out_specs=pl.BlockSpec((BN, F_H), lambda i: (i, 0)),
        out_shape=jax.ShapeDtypeStruct((NP, F_H), jnp.float32),
        interpret=interpret,
    )(acc, acc, cnt2d, cnt2d, x, root, bias)


def _node2_body(acc0_ref, acc1_ref, cnt0_ref, cnt1_ref, h1_ref, b_ref,
                root_ref, bias_ref, Wp1_ref, bp1_ref, Wp2_ref, bp2_ref,
                Wp3_ref, bp3_ref, out_o_ref, out_r_ref, accg, cntg):
    i = pl.program_id(0)

    @pl.when(i == 0)
    def _():
        accg[...] = jnp.zeros_like(accg)
        cntg[...] = jnp.zeros_like(cntg)

    c = jnp.maximum(cnt0_ref[...] + cnt1_ref[...], 1.0)
    a = (acc0_ref[...] + acc1_ref[...]) / c
    t = a + jnp.dot(h1_ref[...], root_ref[...],
                    preferred_element_type=jnp.float32, precision=lax.Precision.HIGHEST) + bias_ref[...]
    h2 = jnp.where(t > 0.0, t, jnp.exp(t) - 1.0)

    ids = b_ref[0]
    onehot = (lax.broadcasted_iota(jnp.int32, (N_G, BN), 0)
              == ids).astype(jnp.float32)
    accg[...] += jnp.dot(onehot, h2, preferred_element_type=jnp.float32, precision=lax.Precision.HIGHEST)
    cntg[...] += jnp.broadcast_to(
        jnp.sum(onehot, axis=1, keepdims=True), (N_G, F_H))

    @pl.when(i == NBN - 1)
    def _():
        r = accg[...] / jnp.maximum(cntg[...], 1.0)
        o = jnp.maximum(
            jnp.dot(r, Wp1_ref[...],
                    preferred_element_type=jnp.float32, precision=lax.Precision.HIGHEST) + bp1_ref[...], 0.0)
        o = jnp.maximum(
            jnp.dot(o, Wp2_ref[...],
                    preferred_element_type=jnp.float32, precision=lax.Precision.HIGHEST) + bp2_ref[...], 0.0)
        o3 = jnp.dot(o, Wp3_ref[...],
                     preferred_element_type=jnp.float32, precision=lax.Precision.HIGHEST) + bp3_ref[...]
        out_o_ref[...] = o3
        out_r_ref[...] = r


def _node2_call(acc, cnt2d, h1, batch3, rootl, biasl, Wp1, bp1, Wp2, bp2,
                Wp3, bp3, interpret=False):
    return pl.pallas_call(
        _node2_body,
        grid=(NBN,),
        in_specs=[
            pl.BlockSpec((BN, F_H), lambda i: (i, 0)),
            pl.BlockSpec((BN, F_H), lambda i: (i + NBN, 0)),
            pl.BlockSpec((BN, 1), lambda i: (i, 0)),
            pl.BlockSpec((BN, 1), lambda i: (i + NBN, 0)),
            pl.BlockSpec((BN, F_H), lambda i: (i, 0)),
            pl.BlockSpec((1, 1, BN), lambda i: (i, 0, 0)),
            pl.BlockSpec((F_H, F_H), lambda i: (0, 0)),
            pl.BlockSpec((1, F_H), lambda i: (0, 0)),
            pl.BlockSpec((F_H, F_H), lambda i: (0, 0)),
            pl.BlockSpec((1, F_H), lambda i: (0, 0)),
            pl.BlockSpec((F_H, F_H // 2), lambda i: (0, 0)),
            pl.BlockSpec((1, F_H // 2), lambda i: (0, 0)),
            pl.BlockSpec((F_H // 2, 1), lambda i: (0, 0)),
            pl.BlockSpec((1, 1), lambda i: (0, 0)),
        ],
        out_specs=[
            pl.BlockSpec((N_G, 1), lambda i: (0, 0)),
            pl.BlockSpec((N_G, F_H), lambda i: (0, 0)),
        ],
        out_shape=[
            jax.ShapeDtypeStruct((N_G, 1), jnp.float32),
            jax.ShapeDtypeStruct((N_G, F_H), jnp.float32),
        ],
        scratch_shapes=[
            pltpu.VMEM((N_G, F_H), jnp.float32),
            pltpu.VMEM((N_G, F_H), jnp.float32),
        ],
        interpret=interpret,
    )(acc, acc, cnt2d, cnt2d, h1, batch3, rootl, biasl, Wp1, bp1, Wp2, bp2,
      Wp3, bp3)


def kernel(x, edge_index, edge_attr, batch, W1a, b1a, W1b, b1b, root1, bias1,
           Wla, bla, Wlb, blb, rootl, biasl, Wp1, bp1, Wp2, bp2, Wp3, bp3):
    x = jnp.pad(x.astype(jnp.float32), ((0, NP - N_NODES), (0, 0)))
    ea = jnp.pad(edge_attr.astype(jnp.float32),
                 ((0, EP - N_EDGES), (0, 0)))
    src = edge_index[0].astype(jnp.int32)
    dst = edge_index[1].astype(jnp.int32)
    pad_e = EP - N_EDGES
    src_r = jnp.concatenate(
        [src, jnp.zeros((pad_e,), jnp.int32)]).reshape(NW, NCH, CHUNK)
    dst_r = jnp.concatenate(
        [dst, jnp.full((pad_e,), N_NODES, jnp.int32)]).reshape(NW, NCH, CHUNK)
    batch3 = jnp.concatenate(
        [batch.astype(jnp.int32),
         jnp.full((NP - N_NODES,), N_G, jnp.int32)]).reshape(NBN, 1, BN)

    z2 = jnp.zeros((NP, F_H), jnp.float32)
    z1 = jnp.zeros((NP,), jnp.float32)
    ones = jnp.ones((CHUNK,), jnp.float32)

    xs = _sc_gather(x, src_r)
    msg1 = _msg_call(ea, xs, W1a, b1a, W1b, b1b)
    acc1, cnt = _sc_scatter(msg1, dst_r, z2, z1, ones, with_cnt=True)
    cnt2d = cnt.reshape(NC * NP, 1)
    h1 = _node1_call(acc1, cnt2d, x, root1, bias1.reshape(1, F_H))

    hs = _sc_gather(h1, src_r)
    msg2 = _msg_call(ea, hs, Wla, bla, Wlb, blb)
    acc2 = _sc_scatter(msg2, dst_r, z2, z1, ones, with_cnt=False)
    out_o, out_r = _node2_call(
        acc2, cnt2d, h1, batch3, rootl, biasl.reshape(1, F_H), Wp1,
        bp1.reshape(1, F_H), Wp2, bp2.reshape(1, F_H // 2), Wp3,
        bp3.reshape(1, 1))
    return out_o.reshape(N_G), out_r


# speedup vs baseline: 2.3039x; 1.6446x over previous
import functools

import jax
import jax.numpy as jnp
from jax import lax
from jax.experimental import pallas as pl
from jax.experimental.pallas import tpu as pltpu
from jax.experimental.pallas import tpu_sc as plsc

N_NODES = 10000
N_EDGES = 160000
F_IN = 32
F_H = 32
F_EF = 16
N_G = 64

NP = 10240
EP = 163840

NC = 2
NS = 16
NW = NC * NS
EPW = EP // NW
CHUNK = 128
NCH = EPW // CHUNK
RPT = NP // NS

_SC_MESH = dict(core_axis_name="c", subcore_axis_name="s")


def _sc_gather(table, idx_r):
    F = table.shape[1]

    @functools.partial(
        pl.kernel,
        out_type=jax.ShapeDtypeStruct((EP, F), jnp.float32),
        scratch_types=[
            pltpu.VMEM((NCH, CHUNK), jnp.int32),
            pltpu.VMEM((CHUNK, F), jnp.float32),
            pltpu.SemaphoreType.DMA,
        ],
        mesh=plsc.VectorSubcoreMesh(**_SC_MESH),
        compiler_params=pltpu.CompilerParams(use_tc_tiling_on_sc=False),
    )
    def gk(table_hbm, idx_hbm, out_hbm, idx_v, rows_v, sem):
        cid = lax.axis_index("c")
        sid = lax.axis_index("s")
        wid = cid * NS + sid
        base = wid * EPW
        pltpu.sync_copy(idx_hbm.at[wid], idx_v)

        def body(j, carry):
            off = pl.multiple_of(base + j * CHUNK, 8)
            pltpu.async_copy(table_hbm.at[idx_v.at[j]], rows_v, sem).wait()
            pltpu.sync_copy(rows_v, out_hbm.at[pl.ds(off, CHUNK)])
            return carry

        lax.fori_loop(0, NCH, body, 0)

    return gk(table, idx_r)


def _sc_scatter(msg, dst_r, zeros2d, zeros1d, ones1d, with_cnt):
    F = msg.shape[1]
    outs = [jax.ShapeDtypeStruct((NC * NP, F), jnp.float32)]
    scratch = [
        pltpu.VMEM((NCH, CHUNK), jnp.int32),
        pltpu.VMEM((CHUNK, F), jnp.float32),
        pltpu.VMEM_SHARED((NP, F), jnp.float32),
    ]
    if with_cnt:
        outs.append(jax.ShapeDtypeStruct((NC * NP,), jnp.float32))
        scratch += [
            pltpu.VMEM((CHUNK,), jnp.float32),
            pltpu.VMEM_SHARED((NP,), jnp.float32),
        ]

    @functools.partial(
        pl.kernel,
        out_type=tuple(outs) if with_cnt else outs[0],
        scratch_types=scratch,
        mesh=plsc.VectorSubcoreMesh(**_SC_MESH),
        compiler_params=pltpu.CompilerParams(use_tc_tiling_on_sc=False),
    )
    def sk(msg_hbm, dst_hbm, z2_hbm, z1_hbm, ones_hbm, *refs):
        if with_cnt:
            acc_hbm, cnt_hbm, idx_v, msg_v, acc_sh, ones_v, cnt_sh = refs
        else:
            acc_hbm, idx_v, msg_v, acc_sh = refs
        cid = lax.axis_index("c")
        sid = lax.axis_index("s")
        wid = cid * NS + sid
        base = wid * EPW
        zoff = pl.multiple_of(sid * RPT, 8)

        pltpu.sync_copy(z2_hbm.at[pl.ds(zoff, RPT)],
                        acc_sh.at[pl.ds(zoff, RPT)])
        if with_cnt:
            @pl.when(sid == 0)
            def _():
                pltpu.sync_copy(z1_hbm, cnt_sh)
            pltpu.sync_copy(ones_hbm, ones_v)
        pltpu.sync_copy(dst_hbm.at[wid], idx_v)
        plsc.subcore_barrier()

        def body(j, carry):
            off = pl.multiple_of(base + j * CHUNK, 8)
            pltpu.sync_copy(msg_hbm.at[pl.ds(off, CHUNK)], msg_v)
            pltpu.sync_copy(msg_v, acc_sh.at[idx_v.at[j]], add=True)
            if with_cnt:
                pltpu.sync_copy(ones_v, cnt_sh.at[idx_v.at[j]], add=True)
            return carry

        lax.fori_loop(0, NCH, body, 0)
        plsc.subcore_barrier()

        doff = pl.multiple_of(cid * NP + sid * RPT, 8)
        pltpu.sync_copy(acc_sh.at[pl.ds(zoff, RPT)],
                        acc_hbm.at[pl.ds(doff, RPT)])
        if with_cnt:
            @pl.when(sid == 0)
            def _():
                coff = pl.multiple_of(cid * NP, 8)
                pltpu.sync_copy(cnt_sh, cnt_hbm.at[pl.ds(coff, NP)])

    return sk(msg, dst_r, zeros2d, zeros1d, ones1d)


BE = 1024


def _msg_body(ea_ref, xs_ref, Wa_ref, ba_ref, W2w_ref, S_ref, Bb_ref,
              out_ref):
    h = jnp.maximum(
        jnp.dot(ea_ref[...], Wa_ref[...],
                preferred_element_type=jnp.float32,
                precision=lax.Precision.HIGHEST) + ba_ref[...], 0.0)
    T = jnp.dot(xs_ref[...], W2w_ref[...],
                preferred_element_type=jnp.float32,
                precision=lax.Precision.HIGHEST)
    Hxt = jnp.concatenate([h] * F_H, axis=1)
    msg = jnp.dot(T * Hxt, S_ref[...], preferred_element_type=jnp.float32)
    msg = msg + jnp.dot(xs_ref[...], Bb_ref[...],
                        preferred_element_type=jnp.float32,
                        precision=lax.Precision.HIGHEST)
    out_ref[...] = msg


def _msg_call(ea, xs, Wa, ba, Wb, bb, interpret=False):
    din = xs.shape[1]
    dk = F_H * F_H
    W2w = jnp.transpose(Wb.reshape(F_H, din, F_H), (1, 2, 0)).reshape(din, dk)
    S = jnp.kron(jnp.eye(F_H, dtype=jnp.float32),
                 jnp.ones((F_H, 1), jnp.float32))
    Bb = bb.reshape(din, F_H)
    return pl.pallas_call(
        _msg_body,
        grid=(EP // BE,),
        in_specs=[
            pl.BlockSpec((BE, F_EF), lambda i: (i, 0)),
            pl.BlockSpec((BE, din), lambda i: (i, 0)),
            pl.BlockSpec((F_EF, F_H), lambda i: (0, 0)),
            pl.BlockSpec((1, F_H), lambda i: (0, 0)),
            pl.BlockSpec((din, dk), lambda i: (0, 0)),
            pl.BlockSpec((dk, F_H), lambda i: (0, 0)),
            pl.BlockSpec((din, F_H), lambda i: (0, 0)),
        ],
        out_specs=pl.BlockSpec((BE, F_H), lambda i: (i, 0)),
        out_shape=jax.ShapeDtypeStruct((EP, F_H), jnp.float32),
        interpret=interpret,
    )(ea, xs, Wa, ba.reshape(1, F_H), W2w, S, Bb)


BN = 2048
NBN = NP // BN


def _node1_body(acc0_ref, acc1_ref, cnt0_ref, cnt1_ref, x_ref, root_ref,
                bias_ref, out_ref):
    c = jnp.maximum(cnt0_ref[...] + cnt1_ref[...], 1.0)
    a = (acc0_ref[...] + acc1_ref[...]) / c
    t = a + jnp.dot(x_ref[...], root_ref[...],
                    preferred_element_type=jnp.float32, precision=lax.Precision.HIGHEST) + bias_ref[...]
    out_ref[...] = jnp.where(t > 0.0, t, jnp.exp(t) - 1.0)


def _node1_call(acc, cnt2d, x, root, bias, interpret=False):
    din = x.shape[1]
    return pl.pallas_call(
        _node1_body,
        grid=(NBN,),
        in_specs=[
            pl.BlockSpec((BN, F_H), lambda i: (i, 0)),
            pl.BlockSpec((BN, F_H), lambda i: (i + NBN, 0)),
            pl.BlockSpec((BN, 1), lambda i: (i, 0)),
            pl.BlockSpec((BN, 1), lambda i: (i + NBN, 0)),
            pl.BlockSpec((BN, din), lambda i: (i, 0)),
            pl.BlockSpec((din, F_H), lambda i: (0, 0)),
            pl.BlockSpec((1, F_H), lambda i: (0, 0)),
        ],
        out_specs=pl.BlockSpec((BN, F_H), lambda i: (i, 0)),
        out_shape=jax.ShapeDtypeStruct((NP, F_H), jnp.float32),
        interpret=interpret,
    )(acc, acc, cnt2d, cnt2d, x, root, bias)


def _node2_body(acc0_ref, acc1_ref, cnt0_ref, cnt1_ref, h1_ref, b_ref,
                root_ref, bias_ref, Wp1_ref, bp1_ref, Wp2_ref, bp2_ref,
                Wp3_ref, bp3_ref, out_o_ref, out_r_ref, accg, cntg):
    i = pl.program_id(0)

    @pl.when(i == 0)
    def _():
        accg[...] = jnp.zeros_like(accg)
        cntg[...] = jnp.zeros_like(cntg)

    c = jnp.maximum(cnt0_ref[...] + cnt1_ref[...], 1.0)
    a = (acc0_ref[...] + acc1_ref[...]) / c
    t = a + jnp.dot(h1_ref[...], root_ref[...],
                    preferred_element_type=jnp.float32, precision=lax.Precision.HIGHEST) + bias_ref[...]
    h2 = jnp.where(t > 0.0, t, jnp.exp(t) - 1.0)

    ids = b_ref[0]
    onehot = (lax.broadcasted_iota(jnp.int32, (N_G, BN), 0)
              == ids).astype(jnp.float32)
    accg[...] += jnp.dot(onehot, h2, preferred_element_type=jnp.float32, precision=lax.Precision.HIGHEST)
    cntg[...] += jnp.broadcast_to(
        jnp.sum(onehot, axis=1, keepdims=True), (N_G, F_H))

    @pl.when(i == NBN - 1)
    def _():
        r = accg[...] / jnp.maximum(cntg[...], 1.0)
        o = jnp.maximum(
            jnp.dot(r, Wp1_ref[...],
                    preferred_element_type=jnp.float32, precision=lax.Precision.HIGHEST) + bp1_ref[...], 0.0)
        o = jnp.maximum(
            jnp.dot(o, Wp2_ref[...],
                    preferred_element_type=jnp.float32, precision=lax.Precision.HIGHEST) + bp2_ref[...], 0.0)
        o3 = jnp.dot(o, Wp3_ref[...],
                     preferred_element_type=jnp.float32, precision=lax.Precision.HIGHEST) + bp3_ref[...]
        out_o_ref[...] = o3
        out_r_ref[...] = r


def _node2_call(acc, cnt2d, h1, batch3, rootl, biasl, Wp1, bp1, Wp2, bp2,
                Wp3, bp3, interpret=False):
    return pl.pallas_call(
        _node2_body,
        grid=(NBN,),
        in_specs=[
            pl.BlockSpec((BN, F_H), lambda i: (i, 0)),
            pl.BlockSpec((BN, F_H), lambda i: (i + NBN, 0)),
            pl.BlockSpec((BN, 1), lambda i: (i, 0)),
            pl.BlockSpec((BN, 1), lambda i: (i + NBN, 0)),
            pl.BlockSpec((BN, F_H), lambda i: (i, 0)),
            pl.BlockSpec((1, 1, BN), lambda i: (i, 0, 0)),
            pl.BlockSpec((F_H, F_H), lambda i: (0, 0)),
            pl.BlockSpec((1, F_H), lambda i: (0, 0)),
            pl.BlockSpec((F_H, F_H), lambda i: (0, 0)),
            pl.BlockSpec((1, F_H), lambda i: (0, 0)),
            pl.BlockSpec((F_H, F_H // 2), lambda i: (0, 0)),
            pl.BlockSpec((1, F_H // 2), lambda i: (0, 0)),
            pl.BlockSpec((F_H // 2, 1), lambda i: (0, 0)),
            pl.BlockSpec((1, 1), lambda i: (0, 0)),
        ],
        out_specs=[
            pl.BlockSpec((N_G, 1), lambda i: (0, 0)),
            pl.BlockSpec((N_G, F_H), lambda i: (0, 0)),
        ],
        out_shape=[
            jax.ShapeDtypeStruct((N_G, 1), jnp.float32),
            jax.ShapeDtypeStruct((N_G, F_H), jnp.float32),
        ],
        scratch_shapes=[
            pltpu.VMEM((N_G, F_H), jnp.float32),
            pltpu.VMEM((N_G, F_H), jnp.float32),
        ],
        interpret=interpret,
    )(acc, acc, cnt2d, cnt2d, h1, batch3, rootl, biasl, Wp1, bp1, Wp2, bp2,
      Wp3, bp3)


def kernel(x, edge_index, edge_attr, batch, W1a, b1a, W1b, b1b, root1, bias1,
           Wla, bla, Wlb, blb, rootl, biasl, Wp1, bp1, Wp2, bp2, Wp3, bp3):
    x = jnp.pad(x.astype(jnp.float32), ((0, NP - N_NODES), (0, 0)))
    ea = jnp.pad(edge_attr.astype(jnp.float32),
                 ((0, EP - N_EDGES), (0, 0)))
    src = edge_index[0].astype(jnp.int32)
    dst = edge_index[1].astype(jnp.int32)
    pad_e = EP - N_EDGES
    src_r = jnp.concatenate(
        [src, jnp.zeros((pad_e,), jnp.int32)]).reshape(NW, NCH, CHUNK)
    dst_r = jnp.concatenate(
        [dst, jnp.full((pad_e,), N_NODES, jnp.int32)]).reshape(NW, NCH, CHUNK)
    batch3 = jnp.concatenate(
        [batch.astype(jnp.int32),
         jnp.full((NP - N_NODES,), N_G, jnp.int32)]).reshape(NBN, 1, BN)

    z2 = jnp.zeros((NP, F_H), jnp.float32)
    z1 = jnp.zeros((NP,), jnp.float32)
    ones = jnp.ones((CHUNK,), jnp.float32)

    xs = _sc_gather(x, src_r)
    msg1 = _msg_call(ea, xs, W1a, b1a, W1b, b1b)
    acc1, cnt = _sc_scatter(msg1, dst_r, z2, z1, ones, with_cnt=True)
    cnt2d = cnt.reshape(NC * NP, 1)
    h1 = _node1_call(acc1, cnt2d, x, root1, bias1.reshape(1, F_H))

    hs = _sc_gather(h1, src_r)
    msg2 = _msg_call(ea, hs, Wla, bla, Wlb, blb)
    acc2 = _sc_scatter(msg2, dst_r, z2, z1, ones, with_cnt=False)
    out_o, out_r = _node2_call(
        acc2, cnt2d, h1, batch3, rootl, biasl.reshape(1, F_H), Wp1,
        bp1.reshape(1, F_H), Wp2, bp2.reshape(1, F_H // 2), Wp3,
        bp3.reshape(1, 1))
    return out_o.reshape(N_G), out_r


# speedup vs baseline: 3.3820x; 1.4680x over previous
import functools

import jax
import jax.numpy as jnp
from jax import lax
from jax.experimental import pallas as pl
from jax.experimental.pallas import tpu as pltpu
from jax.experimental.pallas import tpu_sc as plsc

N_NODES = 10000
N_EDGES = 160000
F_IN = 32
F_H = 32
F_EF = 16
N_G = 64

NP = 10240
EP = 163840

NC = 2
NS = 16
NW = NC * NS
EPW = EP // NW
CHUNK = 128
NCH = EPW // CHUNK
RPT = NP // NS

_SC_MESH = dict(core_axis_name="c", subcore_axis_name="s")


def _sc_gather(table, idx_r):
    F = table.shape[1]

    @functools.partial(
        pl.kernel,
        out_type=jax.ShapeDtypeStruct((EP, F), jnp.float32),
        scratch_types=[
            pltpu.VMEM((NCH, CHUNK), jnp.int32),
            pltpu.VMEM((CHUNK, F), jnp.float32),
            pltpu.SemaphoreType.DMA,
        ],
        mesh=plsc.VectorSubcoreMesh(**_SC_MESH),
        compiler_params=pltpu.CompilerParams(use_tc_tiling_on_sc=False),
    )
    def gk(table_hbm, idx_hbm, out_hbm, idx_v, rows_v, sem):
        cid = lax.axis_index("c")
        sid = lax.axis_index("s")
        wid = cid * NS + sid
        base = wid * EPW
        pltpu.sync_copy(idx_hbm.at[wid], idx_v)

        def body(j, carry):
            off = pl.multiple_of(base + j * CHUNK, 8)
            pltpu.async_copy(table_hbm.at[idx_v.at[j]], rows_v, sem).wait()
            pltpu.sync_copy(rows_v, out_hbm.at[pl.ds(off, CHUNK)])
            return carry

        lax.fori_loop(0, NCH, body, 0)

    return gk(table, idx_r)


def _sc_scatter(msg, dst_r, zeros2d, zeros1d, ones1d, with_cnt):
    F = msg.shape[1]
    outs = [jax.ShapeDtypeStruct((NC * NP, F), jnp.float32)]
    scratch = [
        pltpu.VMEM((NCH, CHUNK), jnp.int32),
        pltpu.VMEM((CHUNK, F), jnp.float32),
        pltpu.VMEM_SHARED((NP, F), jnp.float32),
    ]
    if with_cnt:
        outs.append(jax.ShapeDtypeStruct((NC * NP,), jnp.float32))
        scratch += [
            pltpu.VMEM((CHUNK,), jnp.float32),
            pltpu.VMEM_SHARED((NP,), jnp.float32),
        ]

    @functools.partial(
        pl.kernel,
        out_type=tuple(outs) if with_cnt else outs[0],
        scratch_types=scratch,
        mesh=plsc.VectorSubcoreMesh(**_SC_MESH),
        compiler_params=pltpu.CompilerParams(use_tc_tiling_on_sc=False),
    )
    def sk(msg_hbm, dst_hbm, z2_hbm, z1_hbm, ones_hbm, *refs):
        if with_cnt:
            acc_hbm, cnt_hbm, idx_v, msg_v, acc_sh, ones_v, cnt_sh = refs
        else:
            acc_hbm, idx_v, msg_v, acc_sh = refs
        cid = lax.axis_index("c")
        sid = lax.axis_index("s")
        wid = cid * NS + sid
        base = wid * EPW
        zoff = pl.multiple_of(sid * RPT, 8)

        pltpu.sync_copy(z2_hbm.at[pl.ds(zoff, RPT)],
                        acc_sh.at[pl.ds(zoff, RPT)])
        if with_cnt:
            @pl.when(sid == 0)
            def _():
                pltpu.sync_copy(z1_hbm, cnt_sh)
            pltpu.sync_copy(ones_hbm, ones_v)
        pltpu.sync_copy(dst_hbm.at[wid], idx_v)
        plsc.subcore_barrier()

        def body(j, carry):
            off = pl.multiple_of(base + j * CHUNK, 8)
            pltpu.sync_copy(msg_hbm.at[pl.ds(off, CHUNK)], msg_v)
            pltpu.sync_copy(msg_v, acc_sh.at[idx_v.at[j]], add=True)
            if with_cnt:
                pltpu.sync_copy(ones_v, cnt_sh.at[idx_v.at[j]], add=True)
            return carry

        lax.fori_loop(0, NCH, body, 0)
        plsc.subcore_barrier()

        doff = pl.multiple_of(cid * NP + sid * RPT, 8)
        pltpu.sync_copy(acc_sh.at[pl.ds(zoff, RPT)],
                        acc_hbm.at[pl.ds(doff, RPT)])
        if with_cnt:
            @pl.when(sid == 0)
            def _():
                coff = pl.multiple_of(cid * NP, 8)
                pltpu.sync_copy(cnt_sh, cnt_hbm.at[pl.ds(coff, NP)])

    return sk(msg, dst_r, zeros2d, zeros1d, ones1d)


BE = 1024


def _msg_body(ea_ref, xs_ref, Wa_ref, ba_ref, Wbp_ref, bbp_ref, S_ref,
              out_ref):
    h = jnp.maximum(
        jnp.dot(ea_ref[...], Wa_ref[...],
                preferred_element_type=jnp.float32) + ba_ref[...], 0.0)
    hb = h.astype(jnp.bfloat16)
    U = jnp.dot(hb, Wbp_ref[...],
                preferred_element_type=jnp.float32) + bbp_ref[...]
    Ub = U.astype(jnp.bfloat16).astype(jnp.float32)
    xsb = xs_ref[...].astype(jnp.bfloat16).astype(jnp.float32)
    P = Ub * jnp.concatenate([xsb] * F_H, axis=1)
    Phi = P.astype(jnp.bfloat16)
    Plo = (P - Phi.astype(jnp.float32)).astype(jnp.bfloat16)
    msg = (jnp.dot(Phi, S_ref[...], preferred_element_type=jnp.float32)
           + jnp.dot(Plo, S_ref[...], preferred_element_type=jnp.float32))
    out_ref[...] = msg


def _msg_call(ea, xs, Wa, ba, Wb, bb, interpret=False):
    din = xs.shape[1]
    dk = F_H * F_H
    Wbp = jnp.transpose(Wb.reshape(F_H, din, F_H),
                        (0, 2, 1)).reshape(F_H, dk).astype(jnp.bfloat16)
    bbp = jnp.transpose(bb.reshape(din, F_H)).reshape(1, dk)
    S = jnp.kron(jnp.eye(F_H, dtype=jnp.bfloat16),
                 jnp.ones((F_H, 1), jnp.bfloat16))
    return pl.pallas_call(
        _msg_body,
        grid=(EP // BE,),
        in_specs=[
            pl.BlockSpec((BE, F_EF), lambda i: (i, 0)),
            pl.BlockSpec((BE, din), lambda i: (i, 0)),
            pl.BlockSpec((F_EF, F_H), lambda i: (0, 0)),
            pl.BlockSpec((1, F_H), lambda i: (0, 0)),
            pl.BlockSpec((F_H, dk), lambda i: (0, 0)),
            pl.BlockSpec((1, dk), lambda i: (0, 0)),
            pl.BlockSpec((dk, F_H), lambda i: (0, 0)),
        ],
        out_specs=pl.BlockSpec((BE, F_H), lambda i: (i, 0)),
        out_shape=jax.ShapeDtypeStruct((EP, F_H), jnp.float32),
        interpret=interpret,
    )(ea.astype(jnp.bfloat16), xs, Wa.astype(jnp.bfloat16),
      ba.reshape(1, F_H), Wbp, bbp, S)


BN = 2048
NBN = NP // BN


def _node1_body(acc0_ref, acc1_ref, cnt0_ref, cnt1_ref, x_ref, root_ref,
                bias_ref, out_ref):
    c = jnp.maximum(cnt0_ref[...] + cnt1_ref[...], 1.0)
    a = (acc0_ref[...] + acc1_ref[...]) / c
    t = a + jnp.dot(x_ref[...], root_ref[...],
                    preferred_element_type=jnp.float32) + bias_ref[...]
    out_ref[...] = jnp.where(t > 0.0, t, jnp.exp(t) - 1.0)


def _node1_call(acc, cnt2d, x, root, bias, interpret=False):
    din = x.shape[1]
    return pl.pallas_call(
        _node1_body,
        grid=(NBN,),
        in_specs=[
            pl.BlockSpec((BN, F_H), lambda i: (i, 0)),
            pl.BlockSpec((BN, F_H), lambda i: (i + NBN, 0)),
            pl.BlockSpec((BN, 1), lambda i: (i, 0)),
            pl.BlockSpec((BN, 1), lambda i: (i + NBN, 0)),
            pl.BlockSpec((BN, din), lambda i: (i, 0)),
            pl.BlockSpec((din, F_H), lambda i: (0, 0)),
            pl.BlockSpec((1, F_H), lambda i: (0, 0)),
        ],
        out_specs=pl.BlockSpec((BN, F_H), lambda i: (i, 0)),
        out_shape=jax.ShapeDtypeStruct((NP, F_H), jnp.float32),
        interpret=interpret,
    )(acc, acc, cnt2d, cnt2d, x, root, bias)


def _node2_body(acc0_ref, acc1_ref, cnt0_ref, cnt1_ref, h1_ref, b_ref,
                root_ref, bias_ref, Wp1_ref, bp1_ref, Wp2_ref, bp2_ref,
                Wp3_ref, bp3_ref, out_o_ref, out_r_ref, accg, cntg):
    i = pl.program_id(0)

    @pl.when(i == 0)
    def _():
        accg[...] = jnp.zeros_like(accg)
        cntg[...] = jnp.zeros_like(cntg)

    c = jnp.maximum(cnt0_ref[...] + cnt1_ref[...], 1.0)
    a = (acc0_ref[...] + acc1_ref[...]) / c
    t = a + jnp.dot(h1_ref[...], root_ref[...],
                    preferred_element_type=jnp.float32) + bias_ref[...]
    h2 = jnp.where(t > 0.0, t, jnp.exp(t) - 1.0)

    ids = b_ref[0]
    onehot = (lax.broadcasted_iota(jnp.int32, (N_G, BN), 0)
              == ids).astype(jnp.float32)
    accg[...] += jnp.dot(onehot, h2, preferred_element_type=jnp.float32, precision=lax.Precision.HIGHEST)
    cntg[...] += jnp.broadcast_to(
        jnp.sum(onehot, axis=1, keepdims=True), (N_G, F_H))

    @pl.when(i == NBN - 1)
    def _():
        r = accg[...] / jnp.maximum(cntg[...], 1.0)
        o = jnp.maximum(
            jnp.dot(r, Wp1_ref[...],
                    preferred_element_type=jnp.float32) + bp1_ref[...], 0.0)
        o = jnp.maximum(
            jnp.dot(o, Wp2_ref[...],
                    preferred_element_type=jnp.float32) + bp2_ref[...], 0.0)
        o3 = jnp.dot(o, Wp3_ref[...],
                     preferred_element_type=jnp.float32) + bp3_ref[...]
        out_o_ref[...] = o3
        out_r_ref[...] = r


def _node2_call(acc, cnt2d, h1, batch3, rootl, biasl, Wp1, bp1, Wp2, bp2,
                Wp3, bp3, interpret=False):
    return pl.pallas_call(
        _node2_body,
        grid=(NBN,),
        in_specs=[
            pl.BlockSpec((BN, F_H), lambda i: (i, 0)),
            pl.BlockSpec((BN, F_H), lambda i: (i + NBN, 0)),
            pl.BlockSpec((BN, 1), lambda i: (i, 0)),
            pl.BlockSpec((BN, 1), lambda i: (i + NBN, 0)),
            pl.BlockSpec((BN, F_H), lambda i: (i, 0)),
            pl.BlockSpec((1, 1, BN), lambda i: (i, 0, 0)),
            pl.BlockSpec((F_H, F_H), lambda i: (0, 0)),
            pl.BlockSpec((1, F_H), lambda i: (0, 0)),
            pl.BlockSpec((F_H, F_H), lambda i: (0, 0)),
            pl.BlockSpec((1, F_H), lambda i: (0, 0)),
            pl.BlockSpec((F_H, F_H // 2), lambda i: (0, 0)),
            pl.BlockSpec((1, F_H // 2), lambda i: (0, 0)),
            pl.BlockSpec((F_H // 2, 1), lambda i: (0, 0)),
            pl.BlockSpec((1, 1), lambda i: (0, 0)),
        ],
        out_specs=[
            pl.BlockSpec((N_G, 1), lambda i: (0, 0)),
            pl.BlockSpec((N_G, F_H), lambda i: (0, 0)),
        ],
        out_shape=[
            jax.ShapeDtypeStruct((N_G, 1), jnp.float32),
            jax.ShapeDtypeStruct((N_G, F_H), jnp.float32),
        ],
        scratch_shapes=[
            pltpu.VMEM((N_G, F_H), jnp.float32),
            pltpu.VMEM((N_G, F_H), jnp.float32),
        ],
        interpret=interpret,
    )(acc, acc, cnt2d, cnt2d, h1, batch3, rootl, biasl, Wp1, bp1, Wp2, bp2,
      Wp3, bp3)


def kernel(x, edge_index, edge_attr, batch, W1a, b1a, W1b, b1b, root1, bias1,
           Wla, bla, Wlb, blb, rootl, biasl, Wp1, bp1, Wp2, bp2, Wp3, bp3):
    x = jnp.pad(x.astype(jnp.float32), ((0, NP - N_NODES), (0, 0)))
    ea = jnp.pad(edge_attr.astype(jnp.float32),
                 ((0, EP - N_EDGES), (0, 0)))
    src = edge_index[0].astype(jnp.int32)
    dst = edge_index[1].astype(jnp.int32)
    pad_e = EP - N_EDGES
    src_r = jnp.concatenate(
        [src, jnp.zeros((pad_e,), jnp.int32)]).reshape(NW, NCH, CHUNK)
    dst_r = jnp.concatenate(
        [dst, jnp.full((pad_e,), N_NODES, jnp.int32)]).reshape(NW, NCH, CHUNK)
    batch3 = jnp.concatenate(
        [batch.astype(jnp.int32),
         jnp.full((NP - N_NODES,), N_G, jnp.int32)]).reshape(NBN, 1, BN)

    z2 = jnp.zeros((NP, F_H), jnp.float32)
    z1 = jnp.zeros((NP,), jnp.float32)
    ones = jnp.ones((CHUNK,), jnp.float32)

    xs = _sc_gather(x, src_r)
    msg1 = _msg_call(ea, xs, W1a, b1a, W1b, b1b)
    acc1, cnt = _sc_scatter(msg1, dst_r, z2, z1, ones, with_cnt=True)
    cnt2d = cnt.reshape(NC * NP, 1)
    h1 = _node1_call(acc1, cnt2d, x, root1, bias1.reshape(1, F_H))

    hs = _sc_gather(h1, src_r)
    msg2 = _msg_call(ea, hs, Wla, bla, Wlb, blb)
    acc2 = _sc_scatter(msg2, dst_r, z2, z1, ones, with_cnt=False)
    out_o, out_r = _node2_call(
        acc2, cnt2d, h1, batch3, rootl, biasl.reshape(1, F_H), Wp1,
        bp1.reshape(1, F_H), Wp2, bp2.reshape(1, F_H // 2), Wp3,
        bp3.reshape(1, 1))
    return out_o.reshape(N_G), out_r


# speedup vs baseline: 3.4928x; 1.0327x over previous
import functools

import jax
import jax.numpy as jnp
from jax import lax
from jax.experimental import pallas as pl
from jax.experimental.pallas import tpu as pltpu
from jax.experimental.pallas import tpu_sc as plsc

N_NODES = 10000
N_EDGES = 160000
F_IN = 32
F_H = 32
F_EF = 16
N_G = 64

NP = 10240
EP = 163840

NC = 2
NS = 16
NW = NC * NS
EPW = EP // NW
CHUNK = 128
NCH = EPW // CHUNK
RPT = NP // NS

_SC_MESH = dict(core_axis_name="c", subcore_axis_name="s")


def _sc_gather(table, idx_r):
    F = table.shape[1]
    nch = idx_r.shape[1]
    epw = nch * CHUNK

    @functools.partial(
        pl.kernel,
        out_type=jax.ShapeDtypeStruct((NW * epw, F), jnp.float32),
        scratch_types=[
            pltpu.VMEM((nch, CHUNK), jnp.int32),
            pltpu.VMEM((CHUNK, F), jnp.float32),
            pltpu.SemaphoreType.DMA,
        ],
        mesh=plsc.VectorSubcoreMesh(**_SC_MESH),
        compiler_params=pltpu.CompilerParams(use_tc_tiling_on_sc=False),
    )
    def gk(table_hbm, idx_hbm, out_hbm, idx_v, rows_v, sem):
        cid = lax.axis_index("c")
        sid = lax.axis_index("s")
        wid = cid * NS + sid
        base = wid * epw
        pltpu.sync_copy(idx_hbm.at[wid], idx_v)

        def body(j, carry):
            off = pl.multiple_of(base + j * CHUNK, 8)
            pltpu.async_copy(table_hbm.at[idx_v.at[j]], rows_v, sem).wait()
            pltpu.sync_copy(rows_v, out_hbm.at[pl.ds(off, CHUNK)])
            return carry

        lax.fori_loop(0, nch, body, 0)

    return gk(table, idx_r)


def _sc_scatter(msg, dst_r, zeros2d, zeros1d, ones1d, with_cnt):
    F = msg.shape[1]
    nch = dst_r.shape[1]
    epw = nch * CHUNK
    outs = [jax.ShapeDtypeStruct((NC * NP, F), jnp.float32)]
    scratch = [
        pltpu.VMEM((nch, CHUNK), jnp.int32),
        pltpu.VMEM((CHUNK, F), jnp.float32),
        pltpu.VMEM_SHARED((NP, F), jnp.float32),
    ]
    if with_cnt:
        outs.append(jax.ShapeDtypeStruct((NC * NP,), jnp.float32))
        scratch += [
            pltpu.VMEM((CHUNK,), jnp.float32),
            pltpu.VMEM_SHARED((NP,), jnp.float32),
        ]

    @functools.partial(
        pl.kernel,
        out_type=tuple(outs) if with_cnt else outs[0],
        scratch_types=scratch,
        mesh=plsc.VectorSubcoreMesh(**_SC_MESH),
        compiler_params=pltpu.CompilerParams(use_tc_tiling_on_sc=False),
    )
    def sk(msg_hbm, dst_hbm, z2_hbm, z1_hbm, ones_hbm, *refs):
        if with_cnt:
            acc_hbm, cnt_hbm, idx_v, msg_v, acc_sh, ones_v, cnt_sh = refs
        else:
            acc_hbm, idx_v, msg_v, acc_sh = refs
        cid = lax.axis_index("c")
        sid = lax.axis_index("s")
        wid = cid * NS + sid
        base = wid * epw
        zoff = pl.multiple_of(sid * RPT, 8)

        pltpu.sync_copy(z2_hbm.at[pl.ds(zoff, RPT)],
                        acc_sh.at[pl.ds(zoff, RPT)])
        if with_cnt:
            @pl.when(sid == 0)
            def _():
                pltpu.sync_copy(z1_hbm, cnt_sh)
            pltpu.sync_copy(ones_hbm, ones_v)
        pltpu.sync_copy(dst_hbm.at[wid], idx_v)
        plsc.subcore_barrier()

        def body(j, carry):
            off = pl.multiple_of(base + j * CHUNK, 8)
            pltpu.sync_copy(msg_hbm.at[pl.ds(off, CHUNK)], msg_v)
            pltpu.sync_copy(msg_v, acc_sh.at[idx_v.at[j]], add=True)
            if with_cnt:
                pltpu.sync_copy(ones_v, cnt_sh.at[idx_v.at[j]], add=True)
            return carry

        lax.fori_loop(0, nch, body, 0)
        plsc.subcore_barrier()

        doff = pl.multiple_of(cid * NP + sid * RPT, 8)
        pltpu.sync_copy(acc_sh.at[pl.ds(zoff, RPT)],
                        acc_hbm.at[pl.ds(doff, RPT)])
        if with_cnt:
            @pl.when(sid == 0)
            def _():
                coff = pl.multiple_of(cid * NP, 8)
                pltpu.sync_copy(cnt_sh, cnt_hbm.at[pl.ds(coff, NP)])

    return sk(msg, dst_r, zeros2d, zeros1d, ones1d)


BE = 1024


def _msg_body(ea_ref, xs_ref, Wa_ref, ba_ref, Wbp_ref, bbp_ref, S_ref,
              out_ref):
    h = jnp.maximum(
        jnp.dot(ea_ref[...], Wa_ref[...],
                preferred_element_type=jnp.float32) + ba_ref[...], 0.0)
    hb = h.astype(jnp.bfloat16)
    U = jnp.dot(hb, Wbp_ref[...],
                preferred_element_type=jnp.float32) + bbp_ref[...]
    Ub = U.astype(jnp.bfloat16).astype(jnp.float32)
    xsb = xs_ref[...].astype(jnp.bfloat16).astype(jnp.float32)
    P = Ub * jnp.concatenate([xsb] * F_H, axis=1)
    Phi = P.astype(jnp.bfloat16)
    Plo = (P - Phi.astype(jnp.float32)).astype(jnp.bfloat16)
    msg = (jnp.dot(Phi, S_ref[...], preferred_element_type=jnp.float32)
           + jnp.dot(Plo, S_ref[...], preferred_element_type=jnp.float32))
    out_ref[...] = msg


def _msg_call(ea, xs, Wa, ba, Wb, bb, interpret=False):
    din = xs.shape[1]
    dk = F_H * F_H
    Wbp = jnp.transpose(Wb.reshape(F_H, din, F_H),
                        (0, 2, 1)).reshape(F_H, dk).astype(jnp.bfloat16)
    bbp = jnp.transpose(bb.reshape(din, F_H)).reshape(1, dk)
    S = jnp.kron(jnp.eye(F_H, dtype=jnp.bfloat16),
                 jnp.ones((F_H, 1), jnp.bfloat16))
    return pl.pallas_call(
        _msg_body,
        grid=(ea.shape[0] // BE,),
        in_specs=[
            pl.BlockSpec((BE, F_EF), lambda i: (i, 0)),
            pl.BlockSpec((BE, din), lambda i: (i, 0)),
            pl.BlockSpec((F_EF, F_H), lambda i: (0, 0)),
            pl.BlockSpec((1, F_H), lambda i: (0, 0)),
            pl.BlockSpec((F_H, dk), lambda i: (0, 0)),
            pl.BlockSpec((1, dk), lambda i: (0, 0)),
            pl.BlockSpec((dk, F_H), lambda i: (0, 0)),
        ],
        out_specs=pl.BlockSpec((BE, F_H), lambda i: (i, 0)),
        out_shape=jax.ShapeDtypeStruct((ea.shape[0], F_H), jnp.float32),
        interpret=interpret,
    )(ea.astype(jnp.bfloat16), xs, Wa.astype(jnp.bfloat16),
      ba.reshape(1, F_H), Wbp, bbp, S)


BN = 2048
NBN = NP // BN


def _node1_body(a0_ref, a1_ref, b0_ref, b1_ref, ca0_ref, ca1_ref, cb0_ref,
                cb1_ref, x_ref, root_ref, bias_ref, out_ref):
    c = jnp.maximum(
        ca0_ref[...] + ca1_ref[...] + cb0_ref[...] + cb1_ref[...], 1.0)
    a = (a0_ref[...] + a1_ref[...] + b0_ref[...] + b1_ref[...]) / c
    t = a + jnp.dot(x_ref[...], root_ref[...],
                    preferred_element_type=jnp.float32) + bias_ref[...]
    out_ref[...] = jnp.where(t > 0.0, t, jnp.exp(t) - 1.0)


def _node1_call(accA, accB, cntA2d, cntB2d, x, root, bias, interpret=False):
    din = x.shape[1]
    return pl.pallas_call(
        _node1_body,
        grid=(NBN,),
        in_specs=[
            pl.BlockSpec((BN, F_H), lambda i: (i, 0)),
            pl.BlockSpec((BN, F_H), lambda i: (i + NBN, 0)),
            pl.BlockSpec((BN, F_H), lambda i: (i, 0)),
            pl.BlockSpec((BN, F_H), lambda i: (i + NBN, 0)),
            pl.BlockSpec((BN, 1), lambda i: (i, 0)),
            pl.BlockSpec((BN, 1), lambda i: (i + NBN, 0)),
            pl.BlockSpec((BN, 1), lambda i: (i, 0)),
            pl.BlockSpec((BN, 1), lambda i: (i + NBN, 0)),
            pl.BlockSpec((BN, din), lambda i: (i, 0)),
            pl.BlockSpec((din, F_H), lambda i: (0, 0)),
            pl.BlockSpec((1, F_H), lambda i: (0, 0)),
        ],
        out_specs=pl.BlockSpec((BN, F_H), lambda i: (i, 0)),
        out_shape=jax.ShapeDtypeStruct((NP, F_H), jnp.float32),
        interpret=interpret,
    )(accA, accA, accB, accB, cntA2d, cntA2d, cntB2d, cntB2d, x, root, bias)


def _node2_body(a0_ref, a1_ref, b0_ref, b1_ref, ca0_ref, ca1_ref, cb0_ref,
                cb1_ref, h1_ref, b_ref,
                root_ref, bias_ref, Wp1_ref, bp1_ref, Wp2_ref, bp2_ref,
                Wp3_ref, bp3_ref, out_o_ref, out_r_ref, accg, cntg):
    i = pl.program_id(0)

    @pl.when(i == 0)
    def _():
        accg[...] = jnp.zeros_like(accg)
        cntg[...] = jnp.zeros_like(cntg)

    c = jnp.maximum(
        ca0_ref[...] + ca1_ref[...] + cb0_ref[...] + cb1_ref[...], 1.0)
    a = (a0_ref[...] + a1_ref[...] + b0_ref[...] + b1_ref[...]) / c
    t = a + jnp.dot(h1_ref[...], root_ref[...],
                    preferred_element_type=jnp.float32) + bias_ref[...]
    h2 = jnp.where(t > 0.0, t, jnp.exp(t) - 1.0)

    ids = b_ref[0]
    onehot = (lax.broadcasted_iota(jnp.int32, (N_G, BN), 0)
              == ids).astype(jnp.float32)
    accg[...] += jnp.dot(onehot, h2, preferred_element_type=jnp.float32, precision=lax.Precision.HIGHEST)
    cntg[...] += jnp.broadcast_to(
        jnp.sum(onehot, axis=1, keepdims=True), (N_G, F_H))

    @pl.when(i == NBN - 1)
    def _():
        r = accg[...] / jnp.maximum(cntg[...], 1.0)
        o = jnp.maximum(
            jnp.dot(r, Wp1_ref[...],
                    preferred_element_type=jnp.float32) + bp1_ref[...], 0.0)
        o = jnp.maximum(
            jnp.dot(o, Wp2_ref[...],
                    preferred_element_type=jnp.float32) + bp2_ref[...], 0.0)
        o3 = jnp.dot(o, Wp3_ref[...],
                     preferred_element_type=jnp.float32) + bp3_ref[...]
        out_o_ref[...] = o3
        out_r_ref[...] = r


def _node2_call(accA, accB, cntA2d, cntB2d, h1, batch3, rootl, biasl, Wp1,
                bp1, Wp2, bp2, Wp3, bp3, interpret=False):
    return pl.pallas_call(
        _node2_body,
        grid=(NBN,),
        in_specs=[
            pl.BlockSpec((BN, F_H), lambda i: (i, 0)),
            pl.BlockSpec((BN, F_H), lambda i: (i + NBN, 0)),
            pl.BlockSpec((BN, F_H), lambda i: (i, 0)),
            pl.BlockSpec((BN, F_H), lambda i: (i + NBN, 0)),
            pl.BlockSpec((BN, 1), lambda i: (i, 0)),
            pl.BlockSpec((BN, 1), lambda i: (i + NBN, 0)),
            pl.BlockSpec((BN, 1), lambda i: (i, 0)),
            pl.BlockSpec((BN, 1), lambda i: (i + NBN, 0)),
            pl.BlockSpec((BN, F_H), lambda i: (i, 0)),
            pl.BlockSpec((1, 1, BN), lambda i: (i, 0, 0)),
            pl.BlockSpec((F_H, F_H), lambda i: (0, 0)),
            pl.BlockSpec((1, F_H), lambda i: (0, 0)),
            pl.BlockSpec((F_H, F_H), lambda i: (0, 0)),
            pl.BlockSpec((1, F_H), lambda i: (0, 0)),
            pl.BlockSpec((F_H, F_H // 2), lambda i: (0, 0)),
            pl.BlockSpec((1, F_H // 2), lambda i: (0, 0)),
            pl.BlockSpec((F_H // 2, 1), lambda i: (0, 0)),
            pl.BlockSpec((1, 1), lambda i: (0, 0)),
        ],
        out_specs=[
            pl.BlockSpec((N_G, 1), lambda i: (0, 0)),
            pl.BlockSpec((N_G, F_H), lambda i: (0, 0)),
        ],
        out_shape=[
            jax.ShapeDtypeStruct((N_G, 1), jnp.float32),
            jax.ShapeDtypeStruct((N_G, F_H), jnp.float32),
        ],
        scratch_shapes=[
            pltpu.VMEM((N_G, F_H), jnp.float32),
            pltpu.VMEM((N_G, F_H), jnp.float32),
        ],
        interpret=interpret,
    )(accA, accA, accB, accB, cntA2d, cntA2d, cntB2d, cntB2d, h1, batch3,
      rootl, biasl, Wp1, bp1, Wp2, bp2, Wp3, bp3)


def kernel(x, edge_index, edge_attr, batch, W1a, b1a, W1b, b1b, root1, bias1,
           Wla, bla, Wlb, blb, rootl, biasl, Wp1, bp1, Wp2, bp2, Wp3, bp3):
    x = jnp.pad(x.astype(jnp.float32), ((0, NP - N_NODES), (0, 0)))
    ea = jnp.pad(edge_attr.astype(jnp.float32),
                 ((0, EP - N_EDGES), (0, 0)))
    src = edge_index[0].astype(jnp.int32)
    dst = edge_index[1].astype(jnp.int32)
    pad_e = EP - N_EDGES
    src_r = jnp.concatenate(
        [src, jnp.zeros((pad_e,), jnp.int32)]).reshape(NW, NCH, CHUNK)
    dst_r = jnp.concatenate(
        [dst, jnp.full((pad_e,), N_NODES, jnp.int32)]).reshape(NW, NCH, CHUNK)
    batch3 = jnp.concatenate(
        [batch.astype(jnp.int32),
         jnp.full((NP - N_NODES,), N_G, jnp.int32)]).reshape(NBN, 1, BN)

    z2 = jnp.zeros((NP, F_H), jnp.float32)
    z1 = jnp.zeros((NP,), jnp.float32)
    ones = jnp.ones((CHUNK,), jnp.float32)

    EH = EP // 2
    HCH = NCH // 2
    ea_h = [ea[:EH], ea[EH:]]
    src_h = [src_r.reshape(2, NW, HCH, CHUNK)[0],
             src_r.reshape(2, NW, HCH, CHUNK)[1]]
    dst_h = [dst_r.reshape(2, NW, HCH, CHUNK)[0],
             dst_r.reshape(2, NW, HCH, CHUNK)[1]]

    accs1, cnts1 = [], []
    for hf in range(2):
        xs = _sc_gather(x, src_h[hf])
        msg = _msg_call(ea_h[hf], xs, W1a, b1a, W1b, b1b)
        a, c = _sc_scatter(msg, dst_h[hf], z2, z1, ones, with_cnt=True)
        accs1.append(a)
        cnts1.append(c.reshape(NC * NP, 1))
    h1 = _node1_call(accs1[0], accs1[1], cnts1[0], cnts1[1], x, root1,
                     bias1.reshape(1, F_H))

    accs2 = []
    for hf in range(2):
        hs = _sc_gather(h1, src_h[hf])
        msg = _msg_call(ea_h[hf], hs, Wla, bla, Wlb, blb)
        accs2.append(_sc_scatter(msg, dst_h[hf], z2, z1, ones,
                                 with_cnt=False))
    out_o, out_r = _node2_call(
        accs2[0], accs2[1], cnts1[0], cnts1[1], h1, batch3, rootl,
        biasl.reshape(1, F_H), Wp1, bp1.reshape(1, F_H), Wp2,
        bp2.reshape(1, F_H // 2), Wp3, bp3.reshape(1, 1))
    return out_o.reshape(N_G), out_r


# speedup vs baseline: 3.5881x; 1.0273x over previous
import functools

import jax
import jax.numpy as jnp
from jax import lax
from jax.experimental import pallas as pl
from jax.experimental.pallas import tpu as pltpu
from jax.experimental.pallas import tpu_sc as plsc

N_NODES = 10000
N_EDGES = 160000
F_IN = 32
F_H = 32
F_EF = 16
N_G = 64

NP = 10240
EP = 163840

NC = 2
NS = 16
NW = NC * NS
EPW = EP // NW
CHUNK = 128
NCH = EPW // CHUNK
RPT = NP // NS

_SC_MESH = dict(core_axis_name="c", subcore_axis_name="s")


def _sc_gather(table, idx_r):
    F = table.shape[1]
    nch = idx_r.shape[1]
    epw = nch * CHUNK

    @functools.partial(
        pl.kernel,
        out_type=jax.ShapeDtypeStruct((NW * epw, F), jnp.float32),
        scratch_types=[
            pltpu.VMEM((nch, CHUNK), jnp.int32),
            pltpu.VMEM((CHUNK, F), jnp.float32),
            pltpu.SemaphoreType.DMA,
        ],
        mesh=plsc.VectorSubcoreMesh(**_SC_MESH),
        compiler_params=pltpu.CompilerParams(use_tc_tiling_on_sc=False),
    )
    def gk(table_hbm, idx_hbm, out_hbm, idx_v, rows_v, sem):
        cid = lax.axis_index("c")
        sid = lax.axis_index("s")
        wid = cid * NS + sid
        base = wid * epw
        pltpu.sync_copy(idx_hbm.at[wid], idx_v)

        def body(j, carry):
            off = pl.multiple_of(base + j * CHUNK, 8)
            pltpu.async_copy(table_hbm.at[idx_v.at[j]], rows_v, sem).wait()
            pltpu.sync_copy(rows_v, out_hbm.at[pl.ds(off, CHUNK)])
            return carry

        lax.fori_loop(0, nch, body, 0)

    return gk(table, idx_r)


def _sc_scatter(msg, dst_r, zeros2d, zeros1d, ones1d, with_cnt):
    F = msg.shape[1]
    nch = dst_r.shape[1]
    epw = nch * CHUNK
    outs = [jax.ShapeDtypeStruct((NC * NP, F), jnp.float32)]
    scratch = [
        pltpu.VMEM((nch, CHUNK), jnp.int32),
        pltpu.VMEM((CHUNK, F), jnp.float32),
        pltpu.VMEM_SHARED((NP, F), jnp.float32),
    ]
    if with_cnt:
        outs.append(jax.ShapeDtypeStruct((NC * NP,), jnp.float32))
        scratch += [
            pltpu.VMEM((CHUNK,), jnp.float32),
            pltpu.VMEM_SHARED((NP,), jnp.float32),
        ]

    @functools.partial(
        pl.kernel,
        out_type=tuple(outs) if with_cnt else outs[0],
        scratch_types=scratch,
        mesh=plsc.VectorSubcoreMesh(**_SC_MESH),
        compiler_params=pltpu.CompilerParams(use_tc_tiling_on_sc=False),
    )
    def sk(msg_hbm, dst_hbm, z2_hbm, z1_hbm, ones_hbm, *refs):
        if with_cnt:
            acc_hbm, cnt_hbm, idx_v, msg_v, acc_sh, ones_v, cnt_sh = refs
        else:
            acc_hbm, idx_v, msg_v, acc_sh = refs
        cid = lax.axis_index("c")
        sid = lax.axis_index("s")
        wid = cid * NS + sid
        base = wid * epw
        zoff = pl.multiple_of(sid * RPT, 8)

        pltpu.sync_copy(z2_hbm.at[pl.ds(zoff, RPT)],
                        acc_sh.at[pl.ds(zoff, RPT)])
        if with_cnt:
            @pl.when(sid == 0)
            def _():
                pltpu.sync_copy(z1_hbm, cnt_sh)
            pltpu.sync_copy(ones_hbm, ones_v)
        pltpu.sync_copy(dst_hbm.at[wid], idx_v)
        plsc.subcore_barrier()

        def body(j, carry):
            off = pl.multiple_of(base + j * CHUNK, 8)
            pltpu.sync_copy(msg_hbm.at[pl.ds(off, CHUNK)], msg_v)
            pltpu.sync_copy(msg_v, acc_sh.at[idx_v.at[j]], add=True)
            if with_cnt:
                pltpu.sync_copy(ones_v, cnt_sh.at[idx_v.at[j]], add=True)
            return carry

        lax.fori_loop(0, nch, body, 0)
        plsc.subcore_barrier()

        doff = pl.multiple_of(cid * NP + sid * RPT, 8)
        pltpu.sync_copy(acc_sh.at[pl.ds(zoff, RPT)],
                        acc_hbm.at[pl.ds(doff, RPT)])
        if with_cnt:
            @pl.when(sid == 0)
            def _():
                coff = pl.multiple_of(cid * NP, 8)
                pltpu.sync_copy(cnt_sh, cnt_hbm.at[pl.ds(coff, NP)])

    return sk(msg, dst_r, zeros2d, zeros1d, ones1d)


BE = 2048


def _msg_body(ea_ref, xs_ref, Wa_ref, ba_ref, Wbp_ref, bbp_ref, S_ref,
              out_ref):
    h = jnp.maximum(
        jnp.dot(ea_ref[...], Wa_ref[...],
                preferred_element_type=jnp.float32) + ba_ref[...], 0.0)
    hb = h.astype(jnp.bfloat16)
    U = jnp.dot(hb, Wbp_ref[...],
                preferred_element_type=jnp.float32) + bbp_ref[...]
    Ub = U.astype(jnp.bfloat16).astype(jnp.float32)
    xsb = xs_ref[...].astype(jnp.bfloat16).astype(jnp.float32)
    P = Ub * jnp.concatenate([xsb] * F_H, axis=1)
    Phi = P.astype(jnp.bfloat16)
    Plo = (P - Phi.astype(jnp.float32)).astype(jnp.bfloat16)
    msg = (jnp.dot(Phi, S_ref[...], preferred_element_type=jnp.float32)
           + jnp.dot(Plo, S_ref[...], preferred_element_type=jnp.float32))
    out_ref[...] = msg


def _msg_call(ea, xs, Wa, ba, Wb, bb, interpret=False):
    din = xs.shape[1]
    dk = F_H * F_H
    Wbp = jnp.transpose(Wb.reshape(F_H, din, F_H),
                        (0, 2, 1)).reshape(F_H, dk).astype(jnp.bfloat16)
    bbp = jnp.transpose(bb.reshape(din, F_H)).reshape(1, dk)
    S = jnp.kron(jnp.eye(F_H, dtype=jnp.bfloat16),
                 jnp.ones((F_H, 1), jnp.bfloat16))
    return pl.pallas_call(
        _msg_body,
        grid=(ea.shape[0] // BE,),
        in_specs=[
            pl.BlockSpec((BE, F_EF), lambda i: (i, 0)),
            pl.BlockSpec((BE, din), lambda i: (i, 0)),
            pl.BlockSpec((F_EF, F_H), lambda i: (0, 0)),
            pl.BlockSpec((1, F_H), lambda i: (0, 0)),
            pl.BlockSpec((F_H, dk), lambda i: (0, 0)),
            pl.BlockSpec((1, dk), lambda i: (0, 0)),
            pl.BlockSpec((dk, F_H), lambda i: (0, 0)),
        ],
        out_specs=pl.BlockSpec((BE, F_H), lambda i: (i, 0)),
        out_shape=jax.ShapeDtypeStruct((ea.shape[0], F_H), jnp.float32),
        interpret=interpret,
    )(ea.astype(jnp.bfloat16), xs, Wa.astype(jnp.bfloat16),
      ba.reshape(1, F_H), Wbp, bbp, S)


BN = 2048
NBN = NP // BN


def _node1_body(a0_ref, a1_ref, b0_ref, b1_ref, ca0_ref, ca1_ref, cb0_ref,
                cb1_ref, x_ref, root_ref, bias_ref, out_ref):
    c = jnp.maximum(
        ca0_ref[...] + ca1_ref[...] + cb0_ref[...] + cb1_ref[...], 1.0)
    a = (a0_ref[...] + a1_ref[...] + b0_ref[...] + b1_ref[...]) / c
    t = a + jnp.dot(x_ref[...], root_ref[...],
                    preferred_element_type=jnp.float32) + bias_ref[...]
    out_ref[...] = jnp.where(t > 0.0, t, jnp.exp(t) - 1.0)


def _node1_call(accA, accB, cntA2d, cntB2d, x, root, bias, interpret=False):
    din = x.shape[1]
    return pl.pallas_call(
        _node1_body,
        grid=(NBN,),
        in_specs=[
            pl.BlockSpec((BN, F_H), lambda i: (i, 0)),
            pl.BlockSpec((BN, F_H), lambda i: (i + NBN, 0)),
            pl.BlockSpec((BN, F_H), lambda i: (i, 0)),
            pl.BlockSpec((BN, F_H), lambda i: (i + NBN, 0)),
            pl.BlockSpec((BN, 1), lambda i: (i, 0)),
            pl.BlockSpec((BN, 1), lambda i: (i + NBN, 0)),
            pl.BlockSpec((BN, 1), lambda i: (i, 0)),
            pl.BlockSpec((BN, 1), lambda i: (i + NBN, 0)),
            pl.BlockSpec((BN, din), lambda i: (i, 0)),
            pl.BlockSpec((din, F_H), lambda i: (0, 0)),
            pl.BlockSpec((1, F_H), lambda i: (0, 0)),
        ],
        out_specs=pl.BlockSpec((BN, F_H), lambda i: (i, 0)),
        out_shape=jax.ShapeDtypeStruct((NP, F_H), jnp.float32),
        interpret=interpret,
    )(accA, accA, accB, accB, cntA2d, cntA2d, cntB2d, cntB2d, x, root, bias)


def _node2_body(a0_ref, a1_ref, b0_ref, b1_ref, ca0_ref, ca1_ref, cb0_ref,
                cb1_ref, h1_ref, b_ref,
                root_ref, bias_ref, Wp1_ref, bp1_ref, Wp2_ref, bp2_ref,
                Wp3_ref, bp3_ref, out_o_ref, out_r_ref, accg, cntg):
    i = pl.program_id(0)

    @pl.when(i == 0)
    def _():
        accg[...] = jnp.zeros_like(accg)
        cntg[...] = jnp.zeros_like(cntg)

    c = jnp.maximum(
        ca0_ref[...] + ca1_ref[...] + cb0_ref[...] + cb1_ref[...], 1.0)
    a = (a0_ref[...] + a1_ref[...] + b0_ref[...] + b1_ref[...]) / c
    t = a + jnp.dot(h1_ref[...], root_ref[...],
                    preferred_element_type=jnp.float32) + bias_ref[...]
    h2 = jnp.where(t > 0.0, t, jnp.exp(t) - 1.0)

    ids = b_ref[0]
    onehot = (lax.broadcasted_iota(jnp.int32, (N_G, BN), 0)
              == ids).astype(jnp.float32)
    accg[...] += jnp.dot(onehot, h2, preferred_element_type=jnp.float32, precision=lax.Precision.HIGHEST)
    cntg[...] += jnp.broadcast_to(
        jnp.sum(onehot, axis=1, keepdims=True), (N_G, F_H))

    @pl.when(i == NBN - 1)
    def _():
        r = accg[...] / jnp.maximum(cntg[...], 1.0)
        o = jnp.maximum(
            jnp.dot(r, Wp1_ref[...],
                    preferred_element_type=jnp.float32) + bp1_ref[...], 0.0)
        o = jnp.maximum(
            jnp.dot(o, Wp2_ref[...],
                    preferred_element_type=jnp.float32) + bp2_ref[...], 0.0)
        o3 = jnp.dot(o, Wp3_ref[...],
                     preferred_element_type=jnp.float32) + bp3_ref[...]
        out_o_ref[...] = o3
        out_r_ref[...] = r


def _node2_call(accA, accB, cntA2d, cntB2d, h1, batch3, rootl, biasl, Wp1,
                bp1, Wp2, bp2, Wp3, bp3, interpret=False):
    return pl.pallas_call(
        _node2_body,
        grid=(NBN,),
        in_specs=[
            pl.BlockSpec((BN, F_H), lambda i: (i, 0)),
            pl.BlockSpec((BN, F_H), lambda i: (i + NBN, 0)),
            pl.BlockSpec((BN, F_H), lambda i: (i, 0)),
            pl.BlockSpec((BN, F_H), lambda i: (i + NBN, 0)),
            pl.BlockSpec((BN, 1), lambda i: (i, 0)),
            pl.BlockSpec((BN, 1), lambda i: (i + NBN, 0)),
            pl.BlockSpec((BN, 1), lambda i: (i, 0)),
            pl.BlockSpec((BN, 1), lambda i: (i + NBN, 0)),
            pl.BlockSpec((BN, F_H), lambda i: (i, 0)),
            pl.BlockSpec((1, 1, BN), lambda i: (i, 0, 0)),
            pl.BlockSpec((F_H, F_H), lambda i: (0, 0)),
            pl.BlockSpec((1, F_H), lambda i: (0, 0)),
            pl.BlockSpec((F_H, F_H), lambda i: (0, 0)),
            pl.BlockSpec((1, F_H), lambda i: (0, 0)),
            pl.BlockSpec((F_H, F_H // 2), lambda i: (0, 0)),
            pl.BlockSpec((1, F_H // 2), lambda i: (0, 0)),
            pl.BlockSpec((F_H // 2, 1), lambda i: (0, 0)),
            pl.BlockSpec((1, 1), lambda i: (0, 0)),
        ],
        out_specs=[
            pl.BlockSpec((N_G, 1), lambda i: (0, 0)),
            pl.BlockSpec((N_G, F_H), lambda i: (0, 0)),
        ],
        out_shape=[
            jax.ShapeDtypeStruct((N_G, 1), jnp.float32),
            jax.ShapeDtypeStruct((N_G, F_H), jnp.float32),
        ],
        scratch_shapes=[
            pltpu.VMEM((N_G, F_H), jnp.float32),
            pltpu.VMEM((N_G, F_H), jnp.float32),
        ],
        interpret=interpret,
    )(accA, accA, accB, accB, cntA2d, cntA2d, cntB2d, cntB2d, h1, batch3,
      rootl, biasl, Wp1, bp1, Wp2, bp2, Wp3, bp3)


def kernel(x, edge_index, edge_attr, batch, W1a, b1a, W1b, b1b, root1, bias1,
           Wla, bla, Wlb, blb, rootl, biasl, Wp1, bp1, Wp2, bp2, Wp3, bp3):
    x = jnp.pad(x.astype(jnp.float32), ((0, NP - N_NODES), (0, 0)))
    ea = jnp.pad(edge_attr.astype(jnp.float32),
                 ((0, EP - N_EDGES), (0, 0)))
    src = edge_index[0].astype(jnp.int32)
    dst = edge_index[1].astype(jnp.int32)
    pad_e = EP - N_EDGES
    src_r = jnp.concatenate(
        [src, jnp.zeros((pad_e,), jnp.int32)]).reshape(NW, NCH, CHUNK)
    dst_r = jnp.concatenate(
        [dst, jnp.full((pad_e,), N_NODES, jnp.int32)]).reshape(NW, NCH, CHUNK)
    batch3 = jnp.concatenate(
        [batch.astype(jnp.int32),
         jnp.full((NP - N_NODES,), N_G, jnp.int32)]).reshape(NBN, 1, BN)

    z2 = jnp.zeros((NP, F_H), jnp.float32)
    z1 = jnp.zeros((NP,), jnp.float32)
    ones = jnp.ones((CHUNK,), jnp.float32)

    EH = EP // 2
    HCH = NCH // 2
    ea_h = [ea[:EH], ea[EH:]]
    src_h = [src_r.reshape(2, NW, HCH, CHUNK)[0],
             src_r.reshape(2, NW, HCH, CHUNK)[1]]
    dst_h = [dst_r.reshape(2, NW, HCH, CHUNK)[0],
             dst_r.reshape(2, NW, HCH, CHUNK)[1]]

    accs1, cnts1 = [], []
    for hf in range(2):
        xs = _sc_gather(x, src_h[hf])
        msg = _msg_call(ea_h[hf], xs, W1a, b1a, W1b, b1b)
        a, c = _sc_scatter(msg, dst_h[hf], z2, z1, ones, with_cnt=True)
        accs1.append(a)
        cnts1.append(c.reshape(NC * NP, 1))
    h1 = _node1_call(accs1[0], accs1[1], cnts1[0], cnts1[1], x, root1,
                     bias1.reshape(1, F_H))

    accs2 = []
    for hf in range(2):
        hs = _sc_gather(h1, src_h[hf])
        msg = _msg_call(ea_h[hf], hs, Wla, bla, Wlb, blb)
        accs2.append(_sc_scatter(msg, dst_h[hf], z2, z1, ones,
                                 with_cnt=False))
    out_o, out_r = _node2_call(
        accs2[0], accs2[1], cnts1[0], cnts1[1], h1, batch3, rootl,
        biasl.reshape(1, F_H), Wp1, bp1.reshape(1, F_H), Wp2,
        bp2.reshape(1, F_H // 2), Wp3, bp3.reshape(1, 1))
    return out_o.reshape(N_G), out_r


# speedup vs baseline: 3.6373x; 1.0137x over previous
import functools

import jax
import jax.numpy as jnp
from jax import lax
from jax.experimental import pallas as pl
from jax.experimental.pallas import tpu as pltpu
from jax.experimental.pallas import tpu_sc as plsc

N_NODES = 10000
N_EDGES = 160000
F_IN = 32
F_H = 32
F_EF = 16
N_G = 64

NP = 10240
EP = 163840

NC = 2
NS = 16
NW = NC * NS
EPW = EP // NW
CHUNK = 128
NCH = EPW // CHUNK
RPT = NP // NS

_SC_MESH = dict(core_axis_name="c", subcore_axis_name="s")


def _sc_gather(table, idx_r):
    F = table.shape[1]
    nch = idx_r.shape[1]
    epw = nch * CHUNK

    @functools.partial(
        pl.kernel,
        out_type=jax.ShapeDtypeStruct((NW * epw, F), jnp.float32),
        scratch_types=[
            pltpu.VMEM((nch, CHUNK), jnp.int32),
            pltpu.VMEM((CHUNK, F), jnp.float32),
            pltpu.SemaphoreType.DMA,
        ],
        mesh=plsc.VectorSubcoreMesh(**_SC_MESH),
        compiler_params=pltpu.CompilerParams(use_tc_tiling_on_sc=False),
    )
    def gk(table_hbm, idx_hbm, out_hbm, idx_v, rows_v, sem):
        cid = lax.axis_index("c")
        sid = lax.axis_index("s")
        wid = cid * NS + sid
        base = wid * epw
        pltpu.sync_copy(idx_hbm.at[wid], idx_v)

        def body(j, carry):
            off = pl.multiple_of(base + j * CHUNK, 8)
            pltpu.async_copy(table_hbm.at[idx_v.at[j]], rows_v, sem).wait()
            pltpu.sync_copy(rows_v, out_hbm.at[pl.ds(off, CHUNK)])
            return carry

        lax.fori_loop(0, nch, body, 0)

    return gk(table, idx_r)


def _sc_scatter(msg, dst_r, zeros2d, zeros1d, ones1d, with_cnt):
    F = msg.shape[1]
    nch = dst_r.shape[1]
    epw = nch * CHUNK
    outs = [jax.ShapeDtypeStruct((NC * NP, F), jnp.float32)]
    scratch = [
        pltpu.VMEM((nch, CHUNK), jnp.int32),
        pltpu.VMEM((CHUNK, F), jnp.float32),
        pltpu.VMEM_SHARED((NP, F), jnp.float32),
    ]
    if with_cnt:
        outs.append(jax.ShapeDtypeStruct((NC * NP,), jnp.float32))
        scratch += [
            pltpu.VMEM((CHUNK,), jnp.float32),
            pltpu.VMEM_SHARED((NP,), jnp.float32),
        ]

    @functools.partial(
        pl.kernel,
        out_type=tuple(outs) if with_cnt else outs[0],
        scratch_types=scratch,
        mesh=plsc.VectorSubcoreMesh(**_SC_MESH),
        compiler_params=pltpu.CompilerParams(use_tc_tiling_on_sc=False),
    )
    def sk(msg_hbm, dst_hbm, z2_hbm, z1_hbm, ones_hbm, *refs):
        if with_cnt:
            acc_hbm, cnt_hbm, idx_v, msg_v, acc_sh, ones_v, cnt_sh = refs
        else:
            acc_hbm, idx_v, msg_v, acc_sh = refs
        cid = lax.axis_index("c")
        sid = lax.axis_index("s")
        wid = cid * NS + sid
        base = wid * epw
        zoff = pl.multiple_of(sid * RPT, 8)

        pltpu.sync_copy(z2_hbm.at[pl.ds(zoff, RPT)],
                        acc_sh.at[pl.ds(zoff, RPT)])
        if with_cnt:
            @pl.when(sid == 0)
            def _():
                pltpu.sync_copy(z1_hbm, cnt_sh)
            pltpu.sync_copy(ones_hbm, ones_v)
        pltpu.sync_copy(dst_hbm.at[wid], idx_v)
        plsc.subcore_barrier()

        def body(j, carry):
            off = pl.multiple_of(base + j * CHUNK, 8)
            pltpu.sync_copy(msg_hbm.at[pl.ds(off, CHUNK)], msg_v)
            pltpu.sync_copy(msg_v, acc_sh.at[idx_v.at[j]], add=True)
            if with_cnt:
                pltpu.sync_copy(ones_v, cnt_sh.at[idx_v.at[j]], add=True)
            return carry

        lax.fori_loop(0, nch, body, 0)
        plsc.subcore_barrier()

        doff = pl.multiple_of(cid * NP + sid * RPT, 8)
        pltpu.sync_copy(acc_sh.at[pl.ds(zoff, RPT)],
                        acc_hbm.at[pl.ds(doff, RPT)])
        if with_cnt:
            @pl.when(sid == 0)
            def _():
                coff = pl.multiple_of(cid * NP, 8)
                pltpu.sync_copy(cnt_sh, cnt_hbm.at[pl.ds(coff, NP)])

    return sk(msg, dst_r, zeros2d, zeros1d, ones1d)


BE = 4096


def _msg_body(ea_ref, xs_ref, Wa_ref, ba_ref, Wbp_ref, bbp_ref, S_ref,
              out_ref):
    h = jnp.maximum(
        jnp.dot(ea_ref[...], Wa_ref[...],
                preferred_element_type=jnp.float32) + ba_ref[...], 0.0)
    hb = h.astype(jnp.bfloat16)
    U = jnp.dot(hb, Wbp_ref[...],
                preferred_element_type=jnp.float32) + bbp_ref[...]
    Ub = U.astype(jnp.bfloat16).astype(jnp.float32)
    xsb = xs_ref[...].astype(jnp.bfloat16).astype(jnp.float32)
    P = Ub * jnp.concatenate([xsb] * F_H, axis=1)
    Phi = P.astype(jnp.bfloat16)
    Plo = (P - Phi.astype(jnp.float32)).astype(jnp.bfloat16)
    msg = (jnp.dot(Phi, S_ref[...], preferred_element_type=jnp.float32)
           + jnp.dot(Plo, S_ref[...], preferred_element_type=jnp.float32))
    out_ref[...] = msg


def _msg_call(ea, xs, Wa, ba, Wb, bb, interpret=False):
    din = xs.shape[1]
    dk = F_H * F_H
    Wbp = jnp.transpose(Wb.reshape(F_H, din, F_H),
                        (0, 2, 1)).reshape(F_H, dk).astype(jnp.bfloat16)
    bbp = jnp.transpose(bb.reshape(din, F_H)).reshape(1, dk)
    S = jnp.kron(jnp.eye(F_H, dtype=jnp.bfloat16),
                 jnp.ones((F_H, 1), jnp.bfloat16))
    return pl.pallas_call(
        _msg_body,
        grid=(ea.shape[0] // BE,),
        in_specs=[
            pl.BlockSpec((BE, F_EF), lambda i: (i, 0)),
            pl.BlockSpec((BE, din), lambda i: (i, 0)),
            pl.BlockSpec((F_EF, F_H), lambda i: (0, 0)),
            pl.BlockSpec((1, F_H), lambda i: (0, 0)),
            pl.BlockSpec((F_H, dk), lambda i: (0, 0)),
            pl.BlockSpec((1, dk), lambda i: (0, 0)),
            pl.BlockSpec((dk, F_H), lambda i: (0, 0)),
        ],
        out_specs=pl.BlockSpec((BE, F_H), lambda i: (i, 0)),
        out_shape=jax.ShapeDtypeStruct((ea.shape[0], F_H), jnp.float32),
        interpret=interpret,
    )(ea.astype(jnp.bfloat16), xs, Wa.astype(jnp.bfloat16),
      ba.reshape(1, F_H), Wbp, bbp, S)


BN = 2048
NBN = NP // BN


def _node1_body(a0_ref, a1_ref, b0_ref, b1_ref, ca0_ref, ca1_ref, cb0_ref,
                cb1_ref, x_ref, root_ref, bias_ref, out_ref):
    c = jnp.maximum(
        ca0_ref[...] + ca1_ref[...] + cb0_ref[...] + cb1_ref[...], 1.0)
    a = (a0_ref[...] + a1_ref[...] + b0_ref[...] + b1_ref[...]) / c
    t = a + jnp.dot(x_ref[...], root_ref[...],
                    preferred_element_type=jnp.float32) + bias_ref[...]
    out_ref[...] = jnp.where(t > 0.0, t, jnp.exp(t) - 1.0)


def _node1_call(accA, accB, cntA2d, cntB2d, x, root, bias, interpret=False):
    din = x.shape[1]
    return pl.pallas_call(
        _node1_body,
        grid=(NBN,),
        in_specs=[
            pl.BlockSpec((BN, F_H), lambda i: (i, 0)),
            pl.BlockSpec((BN, F_H), lambda i: (i + NBN, 0)),
            pl.BlockSpec((BN, F_H), lambda i: (i, 0)),
            pl.BlockSpec((BN, F_H), lambda i: (i + NBN, 0)),
            pl.BlockSpec((BN, 1), lambda i: (i, 0)),
            pl.BlockSpec((BN, 1), lambda i: (i + NBN, 0)),
            pl.BlockSpec((BN, 1), lambda i: (i, 0)),
            pl.BlockSpec((BN, 1), lambda i: (i + NBN, 0)),
            pl.BlockSpec((BN, din), lambda i: (i, 0)),
            pl.BlockSpec((din, F_H), lambda i: (0, 0)),
            pl.BlockSpec((1, F_H), lambda i: (0, 0)),
        ],
        out_specs=pl.BlockSpec((BN, F_H), lambda i: (i, 0)),
        out_shape=jax.ShapeDtypeStruct((NP, F_H), jnp.float32),
        interpret=interpret,
    )(accA, accA, accB, accB, cntA2d, cntA2d, cntB2d, cntB2d, x, root, bias)


def _node2_body(a0_ref, a1_ref, b0_ref, b1_ref, ca0_ref, ca1_ref, cb0_ref,
                cb1_ref, h1_ref, b_ref,
                root_ref, bias_ref, Wp1_ref, bp1_ref, Wp2_ref, bp2_ref,
                Wp3_ref, bp3_ref, out_o_ref, out_r_ref, accg, cntg):
    i = pl.program_id(0)

    @pl.when(i == 0)
    def _():
        accg[...] = jnp.zeros_like(accg)
        cntg[...] = jnp.zeros_like(cntg)

    c = jnp.maximum(
        ca0_ref[...] + ca1_ref[...] + cb0_ref[...] + cb1_ref[...], 1.0)
    a = (a0_ref[...] + a1_ref[...] + b0_ref[...] + b1_ref[...]) / c
    t = a + jnp.dot(h1_ref[...], root_ref[...],
                    preferred_element_type=jnp.float32) + bias_ref[...]
    h2 = jnp.where(t > 0.0, t, jnp.exp(t) - 1.0)

    ids = b_ref[0]
    onehot = (lax.broadcasted_iota(jnp.int32, (N_G, BN), 0)
              == ids).astype(jnp.float32)
    accg[...] += jnp.dot(onehot, h2, preferred_element_type=jnp.float32, precision=lax.Precision.HIGHEST)
    cntg[...] += jnp.broadcast_to(
        jnp.sum(onehot, axis=1, keepdims=True), (N_G, F_H))

    @pl.when(i == NBN - 1)
    def _():
        r = accg[...] / jnp.maximum(cntg[...], 1.0)
        o = jnp.maximum(
            jnp.dot(r, Wp1_ref[...],
                    preferred_element_type=jnp.float32) + bp1_ref[...], 0.0)
        o = jnp.maximum(
            jnp.dot(o, Wp2_ref[...],
                    preferred_element_type=jnp.float32) + bp2_ref[...], 0.0)
        o3 = jnp.dot(o, Wp3_ref[...],
                     preferred_element_type=jnp.float32) + bp3_ref[...]
        out_o_ref[...] = o3
        out_r_ref[...] = r


def _node2_call(accA, accB, cntA2d, cntB2d, h1, batch3, rootl, biasl, Wp1,
                bp1, Wp2, bp2, Wp3, bp3, interpret=False):
    return pl.pallas_call(
        _node2_body,
        grid=(NBN,),
        in_specs=[
            pl.BlockSpec((BN, F_H), lambda i: (i, 0)),
            pl.BlockSpec((BN, F_H), lambda i: (i + NBN, 0)),
            pl.BlockSpec((BN, F_H), lambda i: (i, 0)),
            pl.BlockSpec((BN, F_H), lambda i: (i + NBN, 0)),
            pl.BlockSpec((BN, 1), lambda i: (i, 0)),
            pl.BlockSpec((BN, 1), lambda i: (i + NBN, 0)),
            pl.BlockSpec((BN, 1), lambda i: (i, 0)),
            pl.BlockSpec((BN, 1), lambda i: (i + NBN, 0)),
            pl.BlockSpec((BN, F_H), lambda i: (i, 0)),
            pl.BlockSpec((1, 1, BN), lambda i: (i, 0, 0)),
            pl.BlockSpec((F_H, F_H), lambda i: (0, 0)),
            pl.BlockSpec((1, F_H), lambda i: (0, 0)),
            pl.BlockSpec((F_H, F_H), lambda i: (0, 0)),
            pl.BlockSpec((1, F_H), lambda i: (0, 0)),
            pl.BlockSpec((F_H, F_H // 2), lambda i: (0, 0)),
            pl.BlockSpec((1, F_H // 2), lambda i: (0, 0)),
            pl.BlockSpec((F_H // 2, 1), lambda i: (0, 0)),
            pl.BlockSpec((1, 1), lambda i: (0, 0)),
        ],
        out_specs=[
            pl.BlockSpec((N_G, 1), lambda i: (0, 0)),
            pl.BlockSpec((N_G, F_H), lambda i: (0, 0)),
        ],
        out_shape=[
            jax.ShapeDtypeStruct((N_G, 1), jnp.float32),
            jax.ShapeDtypeStruct((N_G, F_H), jnp.float32),
        ],
        scratch_shapes=[
            pltpu.VMEM((N_G, F_H), jnp.float32),
            pltpu.VMEM((N_G, F_H), jnp.float32),
        ],
        interpret=interpret,
    )(accA, accA, accB, accB, cntA2d, cntA2d, cntB2d, cntB2d, h1, batch3,
      rootl, biasl, Wp1, bp1, Wp2, bp2, Wp3, bp3)


def kernel(x, edge_index, edge_attr, batch, W1a, b1a, W1b, b1b, root1, bias1,
           Wla, bla, Wlb, blb, rootl, biasl, Wp1, bp1, Wp2, bp2, Wp3, bp3):
    x = jnp.pad(x.astype(jnp.float32), ((0, NP - N_NODES), (0, 0)))
    ea = jnp.pad(edge_attr.astype(jnp.float32),
                 ((0, EP - N_EDGES), (0, 0)))
    src = edge_index[0].astype(jnp.int32)
    dst = edge_index[1].astype(jnp.int32)
    pad_e = EP - N_EDGES
    src_r = jnp.concatenate(
        [src, jnp.zeros((pad_e,), jnp.int32)]).reshape(NW, NCH, CHUNK)
    dst_r = jnp.concatenate(
        [dst, jnp.full((pad_e,), N_NODES, jnp.int32)]).reshape(NW, NCH, CHUNK)
    batch3 = jnp.concatenate(
        [batch.astype(jnp.int32),
         jnp.full((NP - N_NODES,), N_G, jnp.int32)]).reshape(NBN, 1, BN)

    z2 = jnp.zeros((NP, F_H), jnp.float32)
    z1 = jnp.zeros((NP,), jnp.float32)
    ones = jnp.ones((CHUNK,), jnp.float32)

    EH = EP // 2
    HCH = NCH // 2
    ea_h = [ea[:EH], ea[EH:]]
    src_h = [src_r.reshape(2, NW, HCH, CHUNK)[0],
             src_r.reshape(2, NW, HCH, CHUNK)[1]]
    dst_h = [dst_r.reshape(2, NW, HCH, CHUNK)[0],
             dst_r.reshape(2, NW, HCH, CHUNK)[1]]

    accs1, cnts1 = [], []
    for hf in range(2):
        xs = _sc_gather(x, src_h[hf])
        msg = _msg_call(ea_h[hf], xs, W1a, b1a, W1b, b1b)
        a, c = _sc_scatter(msg, dst_h[hf], z2, z1, ones, with_cnt=True)
        accs1.append(a)
        cnts1.append(c.reshape(NC * NP, 1))
    h1 = _node1_call(accs1[0], accs1[1], cnts1[0], cnts1[1], x, root1,
                     bias1.reshape(1, F_H))

    accs2 = []
    for hf in range(2):
        hs = _sc_gather(h1, src_h[hf])
        msg = _msg_call(ea_h[hf], hs, Wla, bla, Wlb, blb)
        accs2.append(_sc_scatter(msg, dst_h[hf], z2, z1, ones,
                                 with_cnt=False))
    out_o, out_r = _node2_call(
        accs2[0], accs2[1], cnts1[0], cnts1[1], h1, batch3, rootl,
        biasl.reshape(1, F_H), Wp1, bp1.reshape(1, F_H), Wp2,
        bp2.reshape(1, F_H // 2), Wp3, bp3.reshape(1, 1))
    return out_o.reshape(N_G), out_r


# speedup vs baseline: 3.6499x; 1.0035x over previous
import functools

import jax
import jax.numpy as jnp
from jax import lax
from jax.experimental import pallas as pl
from jax.experimental.pallas import tpu as pltpu
from jax.experimental.pallas import tpu_sc as plsc

N_NODES = 10000
N_EDGES = 160000
F_IN = 32
F_H = 32
F_EF = 16
N_G = 64

NP = 10240
EP = 163840

NC = 2
NS = 16
NW = NC * NS
EPW = EP // NW
CHUNK = 128
NCH = EPW // CHUNK
RPT = NP // NS

_SC_MESH = dict(core_axis_name="c", subcore_axis_name="s")


def _sc_gather(table, idx_r):
    F = table.shape[1]
    nch = idx_r.shape[1]
    epw = nch * CHUNK

    @functools.partial(
        pl.kernel,
        out_type=jax.ShapeDtypeStruct((NW * epw, F), jnp.float32),
        scratch_types=[
            pltpu.VMEM((nch, CHUNK), jnp.int32),
            pltpu.VMEM((CHUNK, F), jnp.float32),
            pltpu.VMEM((CHUNK, F), jnp.float32),
            pltpu.SemaphoreType.DMA,
            pltpu.SemaphoreType.DMA,
        ],
        mesh=plsc.VectorSubcoreMesh(**_SC_MESH),
        compiler_params=pltpu.CompilerParams(use_tc_tiling_on_sc=False),
    )
    def gk(table_hbm, idx_hbm, out_hbm, idx_v, rows_a, rows_b, sem_a, sem_b):
        cid = lax.axis_index("c")
        sid = lax.axis_index("s")
        wid = cid * NS + sid
        base = wid * epw
        pltpu.sync_copy(idx_hbm.at[wid], idx_v)

        pltpu.async_copy(table_hbm.at[idx_v.at[0]], rows_a, sem_a)

        def body(j, carry):
            def step(rows_cur, sem_cur, rows_nxt, sem_nxt):
                @pl.when(j + 1 < nch)
                def _():
                    pltpu.async_copy(table_hbm.at[idx_v.at[j + 1]], rows_nxt,
                                     sem_nxt)
                pltpu.make_async_copy(table_hbm.at[idx_v.at[j]], rows_cur,
                                      sem_cur).wait()
                off = pl.multiple_of(base + j * CHUNK, 8)
                pltpu.sync_copy(rows_cur, out_hbm.at[pl.ds(off, CHUNK)])

            even = lax.rem(j, 2) == 0

            @pl.when(even)
            def _():
                step(rows_a, sem_a, rows_b, sem_b)

            @pl.when(jnp.logical_not(even))
            def _():
                step(rows_b, sem_b, rows_a, sem_a)

            return carry

        lax.fori_loop(0, nch, body, 0)

    return gk(table, idx_r)


def _sc_scatter(msg, dst_r, zeros2d, zeros1d, ones1d, with_cnt):
    F = msg.shape[1]
    nch = dst_r.shape[1]
    epw = nch * CHUNK
    outs = [jax.ShapeDtypeStruct((NC * NP, F), jnp.float32)]
    scratch = [
        pltpu.VMEM((nch, CHUNK), jnp.int32),
        pltpu.VMEM((CHUNK, F), jnp.float32),
        pltpu.VMEM_SHARED((NP, F), jnp.float32),
    ]
    if with_cnt:
        outs.append(jax.ShapeDtypeStruct((NC * NP,), jnp.float32))
        scratch += [
            pltpu.VMEM((CHUNK,), jnp.float32),
            pltpu.VMEM_SHARED((NP,), jnp.float32),
        ]

    @functools.partial(
        pl.kernel,
        out_type=tuple(outs) if with_cnt else outs[0],
        scratch_types=scratch,
        mesh=plsc.VectorSubcoreMesh(**_SC_MESH),
        compiler_params=pltpu.CompilerParams(use_tc_tiling_on_sc=False),
    )
    def sk(msg_hbm, dst_hbm, z2_hbm, z1_hbm, ones_hbm, *refs):
        if with_cnt:
            acc_hbm, cnt_hbm, idx_v, msg_v, acc_sh, ones_v, cnt_sh = refs
        else:
            acc_hbm, idx_v, msg_v, acc_sh = refs
        cid = lax.axis_index("c")
        sid = lax.axis_index("s")
        wid = cid * NS + sid
        base = wid * epw
        zoff = pl.multiple_of(sid * RPT, 8)

        pltpu.sync_copy(z2_hbm.at[pl.ds(zoff, RPT)],
                        acc_sh.at[pl.ds(zoff, RPT)])
        if with_cnt:
            @pl.when(sid == 0)
            def _():
                pltpu.sync_copy(z1_hbm, cnt_sh)
            pltpu.sync_copy(ones_hbm, ones_v)
        pltpu.sync_copy(dst_hbm.at[wid], idx_v)
        plsc.subcore_barrier()

        def body(j, carry):
            off = pl.multiple_of(base + j * CHUNK, 8)
            pltpu.sync_copy(msg_hbm.at[pl.ds(off, CHUNK)], msg_v)
            pltpu.sync_copy(msg_v, acc_sh.at[idx_v.at[j]], add=True)
            if with_cnt:
                pltpu.sync_copy(ones_v, cnt_sh.at[idx_v.at[j]], add=True)
            return carry

        lax.fori_loop(0, nch, body, 0)
        plsc.subcore_barrier()

        doff = pl.multiple_of(cid * NP + sid * RPT, 8)
        pltpu.sync_copy(acc_sh.at[pl.ds(zoff, RPT)],
                        acc_hbm.at[pl.ds(doff, RPT)])
        if with_cnt:
            @pl.when(sid == 0)
            def _():
                coff = pl.multiple_of(cid * NP, 8)
                pltpu.sync_copy(cnt_sh, cnt_hbm.at[pl.ds(coff, NP)])

    return sk(msg, dst_r, zeros2d, zeros1d, ones1d)


BE = 4096


def _msg_body(ea_ref, xs_ref, Wa_ref, ba_ref, Wbp_ref, bbp_ref, S_ref,
              out_ref):
    h = jnp.maximum(
        jnp.dot(ea_ref[...], Wa_ref[...],
                preferred_element_type=jnp.float32) + ba_ref[...], 0.0)
    hb = h.astype(jnp.bfloat16)
    U = jnp.dot(hb, Wbp_ref[...],
                preferred_element_type=jnp.float32) + bbp_ref[...]
    Ub = U.astype(jnp.bfloat16).astype(jnp.float32)
    xsb = xs_ref[...].astype(jnp.bfloat16).astype(jnp.float32)
    P = Ub * jnp.concatenate([xsb] * F_H, axis=1)
    Phi = P.astype(jnp.bfloat16)
    Plo = (P - Phi.astype(jnp.float32)).astype(jnp.bfloat16)
    msg = (jnp.dot(Phi, S_ref[...], preferred_element_type=jnp.float32)
           + jnp.dot(Plo, S_ref[...], preferred_element_type=jnp.float32))
    out_ref[...] = msg


def _msg_call(ea, xs, Wa, ba, Wb, bb, interpret=False):
    din = xs.shape[1]
    dk = F_H * F_H
    Wbp = jnp.transpose(Wb.reshape(F_H, din, F_H),
                        (0, 2, 1)).reshape(F_H, dk).astype(jnp.bfloat16)
    bbp = jnp.transpose(bb.reshape(din, F_H)).reshape(1, dk)
    S = jnp.kron(jnp.eye(F_H, dtype=jnp.bfloat16),
                 jnp.ones((F_H, 1), jnp.bfloat16))
    return pl.pallas_call(
        _msg_body,
        grid=(ea.shape[0] // BE,),
        in_specs=[
            pl.BlockSpec((BE, F_EF), lambda i: (i, 0)),
            pl.BlockSpec((BE, din), lambda i: (i, 0)),
            pl.BlockSpec((F_EF, F_H), lambda i: (0, 0)),
            pl.BlockSpec((1, F_H), lambda i: (0, 0)),
            pl.BlockSpec((F_H, dk), lambda i: (0, 0)),
            pl.BlockSpec((1, dk), lambda i: (0, 0)),
            pl.BlockSpec((dk, F_H), lambda i: (0, 0)),
        ],
        out_specs=pl.BlockSpec((BE, F_H), lambda i: (i, 0)),
        out_shape=jax.ShapeDtypeStruct((ea.shape[0], F_H), jnp.float32),
        interpret=interpret,
    )(ea.astype(jnp.bfloat16), xs, Wa.astype(jnp.bfloat16),
      ba.reshape(1, F_H), Wbp, bbp, S)


BN = 2048
NBN = NP // BN


def _node1_body(a0_ref, a1_ref, b0_ref, b1_ref, ca0_ref, ca1_ref, cb0_ref,
                cb1_ref, x_ref, root_ref, bias_ref, out_ref):
    c = jnp.maximum(
        ca0_ref[...] + ca1_ref[...] + cb0_ref[...] + cb1_ref[...], 1.0)
    a = (a0_ref[...] + a1_ref[...] + b0_ref[...] + b1_ref[...]) / c
    t = a + jnp.dot(x_ref[...], root_ref[...],
                    preferred_element_type=jnp.float32) + bias_ref[...]
    out_ref[...] = jnp.where(t > 0.0, t, jnp.exp(t) - 1.0)


def _node1_call(accA, accB, cntA2d, cntB2d, x, root, bias, interpret=False):
    din = x.shape[1]
    return pl.pallas_call(
        _node1_body,
        grid=(NBN,),
        in_specs=[
            pl.BlockSpec((BN, F_H), lambda i: (i, 0)),
            pl.BlockSpec((BN, F_H), lambda i: (i + NBN, 0)),
            pl.BlockSpec((BN, F_H), lambda i: (i, 0)),
            pl.BlockSpec((BN, F_H), lambda i: (i + NBN, 0)),
            pl.BlockSpec((BN, 1), lambda i: (i, 0)),
            pl.BlockSpec((BN, 1), lambda i: (i + NBN, 0)),
            pl.BlockSpec((BN, 1), lambda i: (i, 0)),
            pl.BlockSpec((BN, 1), lambda i: (i + NBN, 0)),
            pl.BlockSpec((BN, din), lambda i: (i, 0)),
            pl.BlockSpec((din, F_H), lambda i: (0, 0)),
            pl.BlockSpec((1, F_H), lambda i: (0, 0)),
        ],
        out_specs=pl.BlockSpec((BN, F_H), lambda i: (i, 0)),
        out_shape=jax.ShapeDtypeStruct((NP, F_H), jnp.float32),
        interpret=interpret,
    )(accA, accA, accB, accB, cntA2d, cntA2d, cntB2d, cntB2d, x, root, bias)


def _node2_body(a0_ref, a1_ref, b0_ref, b1_ref, ca0_ref, ca1_ref, cb0_ref,
                cb1_ref, h1_ref, b_ref,
                root_ref, bias_ref, Wp1_ref, bp1_ref, Wp2_ref, bp2_ref,
                Wp3_ref, bp3_ref, out_o_ref, out_r_ref, accg, cntg):
    i = pl.program_id(0)

    @pl.when(i == 0)
    def _():
        accg[...] = jnp.zeros_like(accg)
        cntg[...] = jnp.zeros_like(cntg)

    c = jnp.maximum(
        ca0_ref[...] + ca1_ref[...] + cb0_ref[...] + cb1_ref[...], 1.0)
    a = (a0_ref[...] + a1_ref[...] + b0_ref[...] + b1_ref[...]) / c
    t = a + jnp.dot(h1_ref[...], root_ref[...],
                    preferred_element_type=jnp.float32) + bias_ref[...]
    h2 = jnp.where(t > 0.0, t, jnp.exp(t) - 1.0)

    ids = b_ref[0]
    onehot = (lax.broadcasted_iota(jnp.int32, (N_G, BN), 0)
              == ids).astype(jnp.float32)
    accg[...] += jnp.dot(onehot, h2, preferred_element_type=jnp.float32, precision=lax.Precision.HIGHEST)
    cntg[...] += jnp.broadcast_to(
        jnp.sum(onehot, axis=1, keepdims=True), (N_G, F_H))

    @pl.when(i == NBN - 1)
    def _():
        r = accg[...] / jnp.maximum(cntg[...], 1.0)
        o = jnp.maximum(
            jnp.dot(r, Wp1_ref[...],
                    preferred_element_type=jnp.float32) + bp1_ref[...], 0.0)
        o = jnp.maximum(
            jnp.dot(o, Wp2_ref[...],
                    preferred_element_type=jnp.float32) + bp2_ref[...], 0.0)
        o3 = jnp.dot(o, Wp3_ref[...],
                     preferred_element_type=jnp.float32) + bp3_ref[...]
        out_o_ref[...] = o3
        out_r_ref[...] = r


def _node2_call(accA, accB, cntA2d, cntB2d, h1, batch3, rootl, biasl, Wp1,
                bp1, Wp2, bp2, Wp3, bp3, interpret=False):
    return pl.pallas_call(
        _node2_body,
        grid=(NBN,),
        in_specs=[
            pl.BlockSpec((BN, F_H), lambda i: (i, 0)),
            pl.BlockSpec((BN, F_H), lambda i: (i + NBN, 0)),
            pl.BlockSpec((BN, F_H), lambda i: (i, 0)),
            pl.BlockSpec((BN, F_H), lambda i: (i + NBN, 0)),
            pl.BlockSpec((BN, 1), lambda i: (i, 0)),
            pl.BlockSpec((BN, 1), lambda i: (i + NBN, 0)),
            pl.BlockSpec((BN, 1), lambda i: (i, 0)),
            pl.BlockSpec((BN, 1), lambda i: (i + NBN, 0)),
            pl.BlockSpec((BN, F_H), lambda i: (i, 0)),
            pl.BlockSpec((1, 1, BN), lambda i: (i, 0, 0)),
            pl.BlockSpec((F_H, F_H), lambda i: (0, 0)),
            pl.BlockSpec((1, F_H), lambda i: (0, 0)),
            pl.BlockSpec((F_H, F_H), lambda i: (0, 0)),
            pl.BlockSpec((1, F_H), lambda i: (0, 0)),
            pl.BlockSpec((F_H, F_H // 2), lambda i: (0, 0)),
            pl.BlockSpec((1, F_H // 2), lambda i: (0, 0)),
            pl.BlockSpec((F_H // 2, 1), lambda i: (0, 0)),
            pl.BlockSpec((1, 1), lambda i: (0, 0)),
        ],
        out_specs=[
            pl.BlockSpec((N_G, 1), lambda i: (0, 0)),
            pl.BlockSpec((N_G, F_H), lambda i: (0, 0)),
        ],
        out_shape=[
            jax.ShapeDtypeStruct((N_G, 1), jnp.float32),
            jax.ShapeDtypeStruct((N_G, F_H), jnp.float32),
        ],
        scratch_shapes=[
            pltpu.VMEM((N_G, F_H), jnp.float32),
            pltpu.VMEM((N_G, F_H), jnp.float32),
        ],
        interpret=interpret,
    )(accA, accA, accB, accB, cntA2d, cntA2d, cntB2d, cntB2d, h1, batch3,
      rootl, biasl, Wp1, bp1, Wp2, bp2, Wp3, bp3)


def kernel(x, edge_index, edge_attr, batch, W1a, b1a, W1b, b1b, root1, bias1,
           Wla, bla, Wlb, blb, rootl, biasl, Wp1, bp1, Wp2, bp2, Wp3, bp3):
    x = jnp.pad(x.astype(jnp.float32), ((0, NP - N_NODES), (0, 0)))
    ea = jnp.pad(edge_attr.astype(jnp.float32),
                 ((0, EP - N_EDGES), (0, 0)))
    src = edge_index[0].astype(jnp.int32)
    dst = edge_index[1].astype(jnp.int32)
    pad_e = EP - N_EDGES
    src_r = jnp.concatenate(
        [src, jnp.zeros((pad_e,), jnp.int32)]).reshape(NW, NCH, CHUNK)
    dst_r = jnp.concatenate(
        [dst, jnp.full((pad_e,), N_NODES, jnp.int32)]).reshape(NW, NCH, CHUNK)
    batch3 = jnp.concatenate(
        [batch.astype(jnp.int32),
         jnp.full((NP - N_NODES,), N_G, jnp.int32)]).reshape(NBN, 1, BN)

    z2 = jnp.zeros((NP, F_H), jnp.float32)
    z1 = jnp.zeros((NP,), jnp.float32)
    ones = jnp.ones((CHUNK,), jnp.float32)

    EH = EP // 2
    HCH = NCH // 2
    ea_h = [ea[:EH], ea[EH:]]
    src_h = [src_r.reshape(2, NW, HCH, CHUNK)[0],
             src_r.reshape(2, NW, HCH, CHUNK)[1]]
    dst_h = [dst_r.reshape(2, NW, HCH, CHUNK)[0],
             dst_r.reshape(2, NW, HCH, CHUNK)[1]]

    accs1, cnts1 = [], []
    for hf in range(2):
        xs = _sc_gather(x, src_h[hf])
        msg = _msg_call(ea_h[hf], xs, W1a, b1a, W1b, b1b)
        a, c = _sc_scatter(msg, dst_h[hf], z2, z1, ones, with_cnt=True)
        accs1.append(a)
        cnts1.append(c.reshape(NC * NP, 1))
    h1 = _node1_call(accs1[0], accs1[1], cnts1[0], cnts1[1], x, root1,
                     bias1.reshape(1, F_H))

    accs2 = []
    for hf in range(2):
        hs = _sc_gather(h1, src_h[hf])
        msg = _msg_call(ea_h[hf], hs, Wla, bla, Wlb, blb)
        accs2.append(_sc_scatter(msg, dst_h[hf], z2, z1, ones,
                                 with_cnt=False))
    out_o, out_r = _node2_call(
        accs2[0], accs2[1], cnts1[0], cnts1[1], h1, batch3, rootl,
        biasl.reshape(1, F_H), Wp1, bp1.reshape(1, F_H), Wp2,
        bp2.reshape(1, F_H // 2), Wp3, bp3.reshape(1, 1))
    return out_o.reshape(N_G), out_r
